# baseline (speedup 1.0000x reference)
import functools

import jax
import jax.numpy as jnp
from jax import lax
from jax.experimental import pallas as pl
from jax.experimental.pallas import tpu as pltpu
from jax.experimental.pallas import tpu_sc as plsc

B, T, L, Lt, V, E, H = 8, 4, 512, 256, 8000, 256, 256
N_SEQ = B + B * T


def _sc_gather(table, idx):
    ni = idx.shape[0]
    info = plsc.get_sparse_core_info()
    nc, ns = info.num_cores, info.num_subcores
    nw = nc * ns
    b_per_w = ni // nw
    n_chunks = 4
    chunk = b_per_w // n_chunks
    mesh = plsc.VectorSubcoreMesh(core_axis_name="c", subcore_axis_name="s")

    @functools.partial(
        pl.kernel,
        out_type=jax.ShapeDtypeStruct((ni, E), jnp.float32),
        mesh=mesh,
        scratch_types=[
            [pltpu.VMEM((chunk,), jnp.int32) for _ in range(n_chunks)],
            pltpu.VMEM((chunk, E), jnp.float32),
            pltpu.SemaphoreType.DMA,
        ],
    )
    def k(table_hbm, idx_hbm, out_hbm, idx_vs, rows_v, sem):
        wid = lax.axis_index("s") * nc + lax.axis_index("c")
        base = wid * b_per_w
        for c in range(n_chunks):
            pltpu.sync_copy(idx_hbm.at[pl.ds(base + c * chunk, chunk)],
                            idx_vs[c])
            pltpu.async_copy(table_hbm.at[idx_vs[c]], rows_v, sem).wait()
            pltpu.sync_copy(rows_v,
                            out_hbm.at[pl.ds(base + c * chunk, chunk)])

    return k(table, idx)


def _dot_t(a, w):
    return lax.dot_general(a, w, (((1,), (1,)), ((), ())),
                           preferred_element_type=jnp.float32)


def _mm_body(a_ref, b_ref, bias_ref, o_ref):
    o_ref[...] = _dot_t(a_ref[...], b_ref[...]) + bias_ref[...]


def _mm_bias(a, w, bias, bm):
    m, k = a.shape
    n = w.shape[0]
    return pl.pallas_call(
        _mm_body,
        grid=(m // bm,),
        in_specs=[
            pl.BlockSpec((bm, k), lambda i: (i, 0)),
            pl.BlockSpec((n, k), lambda i: (0, 0)),
            pl.BlockSpec((1, n), lambda i: (0, 0)),
        ],
        out_specs=pl.BlockSpec((bm, n), lambda i: (i, 0)),
        out_shape=jax.ShapeDtypeStruct((m, n), jnp.float32),
    )(a, w, bias)


def _mm2_body(a1_ref, b1_ref, a2_ref, b2_ref, bias_ref, o_ref):
    o_ref[...] = (_dot_t(a1_ref[...], b1_ref[...])
                  + _dot_t(a2_ref[...], b2_ref[...])
                  + bias_ref[...])


def _mm2_bias(a1, w1, a2, w2, bias, bm):
    m, k = a1.shape
    n = w1.shape[0]
    return pl.pallas_call(
        _mm2_body,
        grid=(m // bm,),
        in_specs=[
            pl.BlockSpec((bm, k), lambda i: (i, 0)),
            pl.BlockSpec((n, k), lambda i: (0, 0)),
            pl.BlockSpec((bm, k), lambda i: (i, 0)),
            pl.BlockSpec((n, k), lambda i: (0, 0)),
            pl.BlockSpec((1, n), lambda i: (0, 0)),
        ],
        out_specs=pl.BlockSpec((bm, n), lambda i: (i, 0)),
        out_shape=jax.ShapeDtypeStruct((m, n), jnp.float32),
    )(a1, w1, a2, w2, bias)


def _lstm_cell(g, h_s, c_s):
    i = jax.nn.sigmoid(g[:, :H])
    f = jax.nn.sigmoid(g[:, H:2 * H])
    gg = jnp.tanh(g[:, 2 * H:3 * H])
    o = jax.nn.sigmoid(g[:, 3 * H:])
    c = f * c_s[...] + i * gg
    h = o * jnp.tanh(c)
    c_s[...] = c
    h_s[...] = h
    return h


def _lstm0_body(igf_ref, igb_ref, wf_ref, wb_ref, hsf_ref, hsb_ref,
                hf, cf, hb, cb):
    t = pl.program_id(0)

    @pl.when(t == 0)
    def _():
        hf[...] = jnp.zeros_like(hf)
        cf[...] = jnp.zeros_like(cf)
        hb[...] = jnp.zeros_like(hb)
        cb[...] = jnp.zeros_like(cb)

    gf = igf_ref[0] + _dot_t(hf[...], wf_ref[...])
    hsf_ref[0] = _lstm_cell(gf, hf, cf)
    gb = igb_ref[0] + _dot_t(hb[...], wb_ref[...])
    hsb_ref[0] = _lstm_cell(gb, hb, cb)


def _lstm_layer0(ig, wf_t, wb_t):
    return pl.pallas_call(
        _lstm0_body,
        grid=(L,),
        in_specs=[
            pl.BlockSpec((1, N_SEQ, 4 * H), lambda t: (t, 0, 0)),
            pl.BlockSpec((1, N_SEQ, 4 * H), lambda t: (L - 1 - t, 0, 1)),
            pl.BlockSpec((4 * H, H), lambda t: (0, 0)),
            pl.BlockSpec((4 * H, H), lambda t: (0, 0)),
        ],
        out_specs=[
            pl.BlockSpec((1, N_SEQ, H), lambda t: (t, 0, 0)),
            pl.BlockSpec((1, N_SEQ, H), lambda t: (L - 1 - t, 0, 0)),
        ],
        out_shape=[
            jax.ShapeDtypeStruct((L, N_SEQ, H), jnp.float32),
            jax.ShapeDtypeStruct((L, N_SEQ, H), jnp.float32),
        ],
        scratch_shapes=[pltpu.VMEM((N_SEQ, H), jnp.float32)] * 4,
    )(ig, ig, wf_t, wb_t)


def _lstm1_body(igf_ref, igb_ref, wf_ref, wb_ref, htf_ref, htb_ref,
                hf, cf, hb, cb):
    t = pl.program_id(0)

    @pl.when(t == 0)
    def _():
        hf[...] = jnp.zeros_like(hf)
        cf[...] = jnp.zeros_like(cf)
        hb[...] = jnp.zeros_like(hb)
        cb[...] = jnp.zeros_like(cb)

    gf = igf_ref[0] + _dot_t(hf[...], wf_ref[...])
    htf_ref[...] = _lstm_cell(gf, hf, cf)
    gb = igb_ref[0] + _dot_t(hb[...], wb_ref[...])
    htb_ref[...] = _lstm_cell(gb, hb, cb)


def _lstm_layer1(ig, wf_t, wb_t):
    return pl.pallas_call(
        _lstm1_body,
        grid=(L,),
        in_specs=[
            pl.BlockSpec((1, N_SEQ, 4 * H), lambda t: (t, 0, 0)),
            pl.BlockSpec((1, N_SEQ, 4 * H), lambda t: (L - 1 - t, 0, 1)),
            pl.BlockSpec((4 * H, H), lambda t: (0, 0)),
            pl.BlockSpec((4 * H, H), lambda t: (0, 0)),
        ],
        out_specs=[
            pl.BlockSpec((N_SEQ, H), lambda t: (0, 0)),
            pl.BlockSpec((N_SEQ, H), lambda t: (0, 0)),
        ],
        out_shape=[
            jax.ShapeDtypeStruct((N_SEQ, H), jnp.float32),
            jax.ShapeDtypeStruct((N_SEQ, H), jnp.float32),
        ],
        scratch_shapes=[pltpu.VMEM((N_SEQ, H), jnp.float32)] * 4,
    )(ig, ig, wf_t, wb_t)


def _bil_body(bin_ref, btmp_ref, hu8_ref, w_ref, bb_ref,
              sal_ref, os_ref):
    nt = B * T
    t1 = jnp.dot(bin_ref[...], w_ref[...],
                 preferred_element_type=jnp.float32)
    s = jnp.sum(t1 * btmp_ref[...], axis=1, keepdims=True) + bb_ref[0, 0]
    sal = jax.nn.sigmoid(s)
    sal_ref[...] = jnp.broadcast_to(sal, (nt, 128))
    es = jnp.exp(sal)
    row = lax.broadcasted_iota(jnp.int32, (nt, nt), 0)
    col = lax.broadcasted_iota(jnp.int32, (nt, nt), 1)
    g = jnp.where(row // T == col // T, 1.0, 0.0).astype(jnp.float32)
    denom = jnp.dot(g, es, preferred_element_type=jnp.float32)
    w = es / denom
    rowp = lax.broadcasted_iota(jnp.int32, (B, nt), 0)
    colp = lax.broadcasted_iota(jnp.int32, (B, nt), 1)
    p = jnp.where(colp // T == rowp, 1.0, 0.0).astype(jnp.float32)
    ts = jnp.dot(p, w * btmp_ref[...], preferred_element_type=jnp.float32)
    os_ref[...] = jnp.concatenate([hu8_ref[...], ts], axis=1)


def _bilinear(bil_in, bil_tmp, hu8, bil_w0, bil_b):
    nt = B * T
    return pl.pallas_call(
        _bil_body,
        in_specs=[
            pl.BlockSpec((nt, 2 * H), lambda: (0, 0)),
            pl.BlockSpec((nt, 2 * H), lambda: (0, 0)),
            pl.BlockSpec((B, 2 * H), lambda: (0, 0)),
            pl.BlockSpec((2 * H, 2 * H), lambda: (0, 0)),
            pl.BlockSpec(memory_space=pltpu.SMEM),
        ],
        out_specs=[
            pl.BlockSpec((nt, 128), lambda: (0, 0)),
            pl.BlockSpec((B, 4 * H), lambda: (0, 0)),
        ],
        out_shape=[
            jax.ShapeDtypeStruct((nt, 128), jnp.float32),
            jax.ShapeDtypeStruct((B, 4 * H), jnp.float32),
        ],
    )(bil_in, bil_tmp, hu8, bil_w0, bil_b.reshape(1, 1))


def _gru_body(gi_ref, u_ref, bh_ref, h0_ref, hs_ref, h):
    t = pl.program_id(0)

    @pl.when(t == 0)
    def _():
        h[...] = h0_ref[...]

    hh = 4 * H
    gh = _dot_t(h[...], u_ref[...]) + bh_ref[...]
    gi = gi_ref[0]
    r = jax.nn.sigmoid(gi[:, :hh] + gh[:, :hh])
    z = jax.nn.sigmoid(gi[:, hh:2 * hh] + gh[:, hh:2 * hh])
    nn_ = jnp.tanh(gi[:, 2 * hh:] + r * gh[:, 2 * hh:])
    hn = (1.0 - z) * nn_ + z * h[...]
    h[...] = hn
    hs_ref[0] = hn


def _gru_seq(gi, u, bh, h0):
    return pl.pallas_call(
        _gru_body,
        grid=(Lt,),
        in_specs=[
            pl.BlockSpec((1, B, 12 * H), lambda t: (t, 0, 0)),
            pl.BlockSpec((12 * H, 4 * H), lambda t: (0, 0)),
            pl.BlockSpec((1, 12 * H), lambda t: (0, 0)),
            pl.BlockSpec((B, 4 * H), lambda t: (0, 0)),
        ],
        out_specs=pl.BlockSpec((1, B, 4 * H), lambda t: (t, 0, 0)),
        out_shape=jax.ShapeDtypeStruct((Lt, B, 4 * H), jnp.float32),
        scratch_shapes=[pltpu.VMEM((B, 4 * H), jnp.float32)],
    )(gi, u, bh, h0)


def _proj_body(a_ref, w_ref, b_ref, o_ref):
    logits = _dot_t(a_ref[...], w_ref[...]) + b_ref[...]
    m = jnp.max(logits, axis=-1, keepdims=True)
    lse = jnp.log(jnp.sum(jnp.exp(logits - m), axis=-1, keepdims=True)) + m
    o_ref[...] = logits - lse


def _out_proj(flat, w, bias):
    bm = 128
    return pl.pallas_call(
        _proj_body,
        grid=(B * Lt // bm,),
        in_specs=[
            pl.BlockSpec((bm, 4 * H), lambda i: (i, 0)),
            pl.BlockSpec((V, 4 * H), lambda i: (0, 0)),
            pl.BlockSpec((1, V), lambda i: (0, 0)),
        ],
        out_specs=pl.BlockSpec((bm, V), lambda i: (i, 0)),
        out_shape=jax.ShapeDtypeStruct((B * Lt, V), jnp.float32),
    )(flat, w, bias)


def kernel(input_ids, target_ids, template_ids, emb,
           l0f_Wih, l0f_Whh, l0f_bih, l0f_bhh,
           l0b_Wih, l0b_Whh, l0b_bih, l0b_bhh,
           l1f_Wih, l1f_Whh, l1f_bih, l1f_bhh,
           l1b_Wih, l1b_Whh, l1b_bih, l1b_bhh,
           bil_W, bil_b, gru_Wih, gru_Whh, gru_bih, gru_bhh,
           out_W, out_b):
    stacked = jnp.concatenate(
        [input_ids, template_ids.reshape(B * T, L)], 0)
    idx_enc = stacked.T.reshape(-1)
    idx_dec = target_ids.T.reshape(-1)
    idx_all = jnp.concatenate([idx_enc, idx_dec]).astype(jnp.int32)
    rows = _sc_gather(emb, idx_all)
    x_enc = rows[:L * N_SEQ]
    x_dec = rows[L * N_SEQ:]

    b0 = jnp.concatenate([l0f_bih + l0f_bhh, l0b_bih + l0b_bhh]).reshape(1, -1)
    w0 = jnp.concatenate([l0f_Wih, l0b_Wih], axis=0)
    ig0 = _mm_bias(x_enc, w0, b0, 1024).reshape(L, N_SEQ, 2 * 4 * H)
    hsf, hsb = _lstm_layer0(ig0, l0f_Whh, l0b_Whh)

    b1 = jnp.concatenate([l1f_bih + l1f_bhh, l1b_bih + l1b_bhh]).reshape(1, -1)
    w1a = jnp.concatenate([l1f_Wih[:, :H], l1b_Wih[:, :H]], axis=0)
    w1b_ = jnp.concatenate([l1f_Wih[:, H:], l1b_Wih[:, H:]], axis=0)
    ig1 = _mm2_bias(hsf.reshape(-1, H), w1a, hsb.reshape(-1, H), w1b_,
                    b1, 1024).reshape(L, N_SEQ, 2 * 4 * H)
    htf, htb = _lstm_layer1(ig1, l1f_Whh, l1b_Whh)

    hidden_u = jnp.concatenate([
        jnp.concatenate([htb[1::2], htb[0::2]], axis=1),
        jnp.concatenate([htf[1::2], htf[0::2]], axis=1)], axis=0)
    bil_in = hidden_u[jnp.repeat(jnp.arange(B), T)]
    bil_tmp = hidden_u[B:]
    sal_pad, out_states = _bilinear(bil_in, bil_tmp, hidden_u[:B],
                                    bil_W[0], bil_b)
    sal_b = sal_pad[:, :1].reshape(B, T, 1)

    bgi = gru_bih.reshape(1, -1)
    gi = _mm_bias(x_dec, gru_Wih, bgi, 512).reshape(Lt, B, 12 * H)
    hs = _gru_seq(gi, gru_Whh, gru_bhh.reshape(1, -1),
                  out_states[::-1])

    flat = hs.transpose(1, 0, 2).reshape(B * Lt, 4 * H)
    lp = _out_proj(flat, out_W, out_b.reshape(1, -1))
    response = lp.reshape(B, Lt, V)
    return (sal_b, response)


# speedup vs baseline: 1.2028x; 1.2028x over previous
import functools

import jax
import jax.numpy as jnp
from jax import lax
from jax.experimental import pallas as pl
from jax.experimental.pallas import tpu as pltpu
from jax.experimental.pallas import tpu_sc as plsc

B, T, L, Lt, V, E, H = 8, 4, 512, 256, 8000, 256, 256
N_SEQ = B + B * T


def _sc_gather(table, idx):
    ni = idx.shape[0]
    info = plsc.get_sparse_core_info()
    nc, ns = info.num_cores, info.num_subcores
    nw = nc * ns
    b_per_w = ni // nw
    n_chunks = 4
    chunk = b_per_w // n_chunks
    mesh = plsc.VectorSubcoreMesh(core_axis_name="c", subcore_axis_name="s")

    @functools.partial(
        pl.kernel,
        out_type=jax.ShapeDtypeStruct((ni, E), jnp.float32),
        mesh=mesh,
        scratch_types=[
            [pltpu.VMEM((chunk,), jnp.int32) for _ in range(n_chunks)],
            pltpu.VMEM((chunk, E), jnp.float32),
            pltpu.SemaphoreType.DMA,
        ],
    )
    def k(table_hbm, idx_hbm, out_hbm, idx_vs, rows_v, sem):
        wid = lax.axis_index("s") * nc + lax.axis_index("c")
        base = wid * b_per_w
        for c in range(n_chunks):
            pltpu.sync_copy(idx_hbm.at[pl.ds(base + c * chunk, chunk)],
                            idx_vs[c])
            pltpu.async_copy(table_hbm.at[idx_vs[c]], rows_v, sem).wait()
            pltpu.sync_copy(rows_v,
                            out_hbm.at[pl.ds(base + c * chunk, chunk)])

    return k(table, idx)


def _dot(a, b):
    return jnp.dot(a.astype(jnp.bfloat16), b,
                   preferred_element_type=jnp.float32)


def _mm_body(a_ref, b_ref, bias_ref, o_ref):
    o_ref[...] = _dot(a_ref[...], b_ref[...]) + bias_ref[...]


def _mm_bias(a, w, bias, bm):
    m, k = a.shape
    n = w.shape[1]
    return pl.pallas_call(
        _mm_body,
        grid=(m // bm,),
        in_specs=[
            pl.BlockSpec((bm, k), lambda i: (i, 0)),
            pl.BlockSpec((k, n), lambda i: (0, 0)),
            pl.BlockSpec((1, n), lambda i: (0, 0)),
        ],
        out_specs=pl.BlockSpec((bm, n), lambda i: (i, 0)),
        out_shape=jax.ShapeDtypeStruct((m, n), jnp.float32),
    )(a, w, bias)


def _mm2_body(a1_ref, b1_ref, a2_ref, b2_ref, bias_ref, o_ref):
    o_ref[...] = (_dot(a1_ref[...], b1_ref[...])
                  + _dot(a2_ref[...], b2_ref[...])
                  + bias_ref[...])


def _mm2_bias(a1, w1, a2, w2, bias, bm):
    m, k = a1.shape
    n = w1.shape[1]
    return pl.pallas_call(
        _mm2_body,
        grid=(m // bm,),
        in_specs=[
            pl.BlockSpec((bm, k), lambda i: (i, 0)),
            pl.BlockSpec((k, n), lambda i: (0, 0)),
            pl.BlockSpec((bm, k), lambda i: (i, 0)),
            pl.BlockSpec((k, n), lambda i: (0, 0)),
            pl.BlockSpec((1, n), lambda i: (0, 0)),
        ],
        out_specs=pl.BlockSpec((bm, n), lambda i: (i, 0)),
        out_shape=jax.ShapeDtypeStruct((m, n), jnp.float32),
    )(a1, w1, a2, w2, bias)


def _lstm_cell(g, h_s, c_s):
    i = jax.nn.sigmoid(g[:, :H])
    f = jax.nn.sigmoid(g[:, H:2 * H])
    gg = jnp.tanh(g[:, 2 * H:3 * H])
    o = jax.nn.sigmoid(g[:, 3 * H:])
    c = f * c_s[...] + i * gg
    h = o * jnp.tanh(c)
    c_s[...] = c
    h_s[...] = h
    return h


def _lstm0_body(igf_ref, igb_ref, wf_ref, wb_ref, hsf_ref, hsb_ref,
                hf, cf, hb, cb):
    t = pl.program_id(0)

    @pl.when(t == 0)
    def _():
        hf[...] = jnp.zeros_like(hf)
        cf[...] = jnp.zeros_like(cf)
        hb[...] = jnp.zeros_like(hb)
        cb[...] = jnp.zeros_like(cb)

    gf = igf_ref[0] + _dot(hf[...], wf_ref[...])
    hsf_ref[0] = _lstm_cell(gf, hf, cf)
    gb = igb_ref[0] + _dot(hb[...], wb_ref[...])
    hsb_ref[0] = _lstm_cell(gb, hb, cb)


def _lstm_layer0(ig, wf_t, wb_t):
    return pl.pallas_call(
        _lstm0_body,
        grid=(L,),
        in_specs=[
            pl.BlockSpec((1, N_SEQ, 4 * H), lambda t: (t, 0, 0)),
            pl.BlockSpec((1, N_SEQ, 4 * H), lambda t: (L - 1 - t, 0, 1)),
            pl.BlockSpec((H, 4 * H), lambda t: (0, 0)),
            pl.BlockSpec((H, 4 * H), lambda t: (0, 0)),
        ],
        out_specs=[
            pl.BlockSpec((1, N_SEQ, H), lambda t: (t, 0, 0)),
            pl.BlockSpec((1, N_SEQ, H), lambda t: (L - 1 - t, 0, 0)),
        ],
        out_shape=[
            jax.ShapeDtypeStruct((L, N_SEQ, H), jnp.float32),
            jax.ShapeDtypeStruct((L, N_SEQ, H), jnp.float32),
        ],
        scratch_shapes=[pltpu.VMEM((N_SEQ, H), jnp.float32)] * 4,
    )(ig, ig, wf_t, wb_t)


def _lstm1_body(igf_ref, igb_ref, wf_ref, wb_ref, htf_ref, htb_ref,
                hf, cf, hb, cb):
    t = pl.program_id(0)

    @pl.when(t == 0)
    def _():
        hf[...] = jnp.zeros_like(hf)
        cf[...] = jnp.zeros_like(cf)
        hb[...] = jnp.zeros_like(hb)
        cb[...] = jnp.zeros_like(cb)

    gf = igf_ref[0] + _dot(hf[...], wf_ref[...])
    htf_ref[...] = _lstm_cell(gf, hf, cf)
    gb = igb_ref[0] + _dot(hb[...], wb_ref[...])
    htb_ref[...] = _lstm_cell(gb, hb, cb)


def _lstm_layer1(ig, wf_t, wb_t):
    return pl.pallas_call(
        _lstm1_body,
        grid=(L,),
        in_specs=[
            pl.BlockSpec((1, N_SEQ, 4 * H), lambda t: (t, 0, 0)),
            pl.BlockSpec((1, N_SEQ, 4 * H), lambda t: (L - 1 - t, 0, 1)),
            pl.BlockSpec((H, 4 * H), lambda t: (0, 0)),
            pl.BlockSpec((H, 4 * H), lambda t: (0, 0)),
        ],
        out_specs=[
            pl.BlockSpec((N_SEQ, H), lambda t: (0, 0)),
            pl.BlockSpec((N_SEQ, H), lambda t: (0, 0)),
        ],
        out_shape=[
            jax.ShapeDtypeStruct((N_SEQ, H), jnp.float32),
            jax.ShapeDtypeStruct((N_SEQ, H), jnp.float32),
        ],
        scratch_shapes=[pltpu.VMEM((N_SEQ, H), jnp.float32)] * 4,
    )(ig, ig, wf_t, wb_t)


def _bil_body(bin_ref, btmp_ref, hu8_ref, w_ref, bb_ref,
              sal_ref, os_ref):
    nt = B * T
    t1 = jnp.dot(bin_ref[...], w_ref[...],
                 preferred_element_type=jnp.float32)
    s = jnp.sum(t1 * btmp_ref[...], axis=1, keepdims=True) + bb_ref[0, 0]
    sal = jax.nn.sigmoid(s)
    sal_ref[...] = jnp.broadcast_to(sal, (nt, 128))
    es = jnp.exp(sal)
    row = lax.broadcasted_iota(jnp.int32, (nt, nt), 0)
    col = lax.broadcasted_iota(jnp.int32, (nt, nt), 1)
    g = jnp.where(row // T == col // T, 1.0, 0.0).astype(jnp.float32)
    denom = jnp.dot(g, es, preferred_element_type=jnp.float32)
    w = es / denom
    rowp = lax.broadcasted_iota(jnp.int32, (B, nt), 0)
    colp = lax.broadcasted_iota(jnp.int32, (B, nt), 1)
    p = jnp.where(colp // T == rowp, 1.0, 0.0).astype(jnp.float32)
    ts = jnp.dot(p, w * btmp_ref[...], preferred_element_type=jnp.float32)
    os_ref[...] = jnp.concatenate([hu8_ref[...], ts], axis=1)


def _bilinear(bil_in, bil_tmp, hu8, bil_w0, bil_b):
    nt = B * T
    return pl.pallas_call(
        _bil_body,
        in_specs=[
            pl.BlockSpec((nt, 2 * H), lambda: (0, 0)),
            pl.BlockSpec((nt, 2 * H), lambda: (0, 0)),
            pl.BlockSpec((B, 2 * H), lambda: (0, 0)),
            pl.BlockSpec((2 * H, 2 * H), lambda: (0, 0)),
            pl.BlockSpec(memory_space=pltpu.SMEM),
        ],
        out_specs=[
            pl.BlockSpec((nt, 128), lambda: (0, 0)),
            pl.BlockSpec((B, 4 * H), lambda: (0, 0)),
        ],
        out_shape=[
            jax.ShapeDtypeStruct((nt, 128), jnp.float32),
            jax.ShapeDtypeStruct((B, 4 * H), jnp.float32),
        ],
    )(bil_in, bil_tmp, hu8, bil_w0, bil_b.reshape(1, 1))


def _gru_body(gi_ref, u_ref, bh_ref, h0_ref, hs_ref, h):
    t = pl.program_id(0)

    @pl.when(t == 0)
    def _():
        h[...] = h0_ref[...]

    hh = 4 * H
    gh = _dot(h[...], u_ref[...]) + bh_ref[...]
    gi = gi_ref[0]
    r = jax.nn.sigmoid(gi[:, :hh] + gh[:, :hh])
    z = jax.nn.sigmoid(gi[:, hh:2 * hh] + gh[:, hh:2 * hh])
    nn_ = jnp.tanh(gi[:, 2 * hh:] + r * gh[:, 2 * hh:])
    hn = (1.0 - z) * nn_ + z * h[...]
    h[...] = hn
    hs_ref[0] = hn


def _gru_seq(gi, u_t, bh, h0):
    return pl.pallas_call(
        _gru_body,
        grid=(Lt,),
        in_specs=[
            pl.BlockSpec((1, B, 12 * H), lambda t: (t, 0, 0)),
            pl.BlockSpec((4 * H, 12 * H), lambda t: (0, 0)),
            pl.BlockSpec((1, 12 * H), lambda t: (0, 0)),
            pl.BlockSpec((B, 4 * H), lambda t: (0, 0)),
        ],
        out_specs=pl.BlockSpec((1, B, 4 * H), lambda t: (t, 0, 0)),
        out_shape=jax.ShapeDtypeStruct((Lt, B, 4 * H), jnp.float32),
        scratch_shapes=[pltpu.VMEM((B, 4 * H), jnp.float32)],
    )(gi, u_t, bh, h0)


def _proj_body(a_ref, w_ref, b_ref, o_ref):
    logits = _dot(a_ref[...], w_ref[...]) + b_ref[...]
    m = jnp.max(logits, axis=-1, keepdims=True)
    lse = jnp.log(jnp.sum(jnp.exp(logits - m), axis=-1, keepdims=True)) + m
    o_ref[...] = logits - lse


def _out_proj(flat, w_t, bias):
    bm = 128
    return pl.pallas_call(
        _proj_body,
        grid=(B * Lt // bm,),
        in_specs=[
            pl.BlockSpec((bm, 4 * H), lambda i: (i, 0)),
            pl.BlockSpec((4 * H, V), lambda i: (0, 0)),
            pl.BlockSpec((1, V), lambda i: (0, 0)),
        ],
        out_specs=pl.BlockSpec((bm, V), lambda i: (i, 0)),
        out_shape=jax.ShapeDtypeStruct((B * Lt, V), jnp.float32),
    )(flat, w_t, bias)


def kernel(input_ids, target_ids, template_ids, emb,
           l0f_Wih, l0f_Whh, l0f_bih, l0f_bhh,
           l0b_Wih, l0b_Whh, l0b_bih, l0b_bhh,
           l1f_Wih, l1f_Whh, l1f_bih, l1f_bhh,
           l1b_Wih, l1b_Whh, l1b_bih, l1b_bhh,
           bil_W, bil_b, gru_Wih, gru_Whh, gru_bih, gru_bhh,
           out_W, out_b):
    stacked = jnp.concatenate(
        [input_ids, template_ids.reshape(B * T, L)], 0)
    idx_enc = stacked.T.reshape(-1)
    idx_dec = target_ids.T.reshape(-1)
    idx_all = jnp.concatenate([idx_enc, idx_dec]).astype(jnp.int32)
    rows = _sc_gather(emb, idx_all)
    x_enc = rows[:L * N_SEQ]
    x_dec = rows[L * N_SEQ:]

    b0 = jnp.concatenate([l0f_bih + l0f_bhh, l0b_bih + l0b_bhh]).reshape(1, -1)
    w0 = jnp.concatenate([l0f_Wih.T, l0b_Wih.T],
                         axis=1).astype(jnp.bfloat16)
    ig0 = _mm_bias(x_enc, w0, b0, 1024).reshape(L, N_SEQ, 2 * 4 * H)
    hsf, hsb = _lstm_layer0(ig0, l0f_Whh.T.astype(jnp.bfloat16),
                            l0b_Whh.T.astype(jnp.bfloat16))

    b1 = jnp.concatenate([l1f_bih + l1f_bhh, l1b_bih + l1b_bhh]).reshape(1, -1)
    w1a = jnp.concatenate([l1f_Wih.T[:H], l1b_Wih.T[:H]],
                          axis=1).astype(jnp.bfloat16)
    w1b_ = jnp.concatenate([l1f_Wih.T[H:], l1b_Wih.T[H:]],
                           axis=1).astype(jnp.bfloat16)
    ig1 = _mm2_bias(hsf.reshape(-1, H), w1a, hsb.reshape(-1, H), w1b_,
                    b1, 1024).reshape(L, N_SEQ, 2 * 4 * H)
    htf, htb = _lstm_layer1(ig1, l1f_Whh.T.astype(jnp.bfloat16),
                            l1b_Whh.T.astype(jnp.bfloat16))

    hidden_u = jnp.concatenate([
        jnp.concatenate([htb[1::2], htb[0::2]], axis=1),
        jnp.concatenate([htf[1::2], htf[0::2]], axis=1)], axis=0)
    bil_in = hidden_u[jnp.repeat(jnp.arange(B), T)]
    bil_tmp = hidden_u[B:]
    sal_pad, out_states = _bilinear(bil_in, bil_tmp, hidden_u[:B],
                                    bil_W[0], bil_b)
    sal_b = sal_pad[:, :1].reshape(B, T, 1)

    bgi = gru_bih.reshape(1, -1)
    gi = _mm_bias(x_dec, gru_Wih.T.astype(jnp.bfloat16),
                  bgi, 512).reshape(Lt, B, 12 * H)
    hs = _gru_seq(gi, gru_Whh.T.astype(jnp.bfloat16), gru_bhh.reshape(1, -1),
                  out_states[::-1])

    flat = hs.transpose(1, 0, 2).reshape(B * Lt, 4 * H)
    lp = _out_proj(flat, out_W.T.astype(jnp.bfloat16),
                   out_b.reshape(1, -1))
    response = lp.reshape(B, Lt, V)
    return (sal_b, response)


# speedup vs baseline: 1.7491x; 1.4542x over previous
import functools

import jax
import jax.numpy as jnp
from jax import lax
from jax.experimental import pallas as pl
from jax.experimental.pallas import tpu as pltpu
from jax.experimental.pallas import tpu_sc as plsc

B, T, L, Lt, V, E, H = 8, 4, 512, 256, 8000, 256, 256
N_SEQ = B + B * T


def _sc_gather(table, idx):
    ni = idx.shape[0]
    info = plsc.get_sparse_core_info()
    nc, ns = info.num_cores, info.num_subcores
    nw = nc * ns
    b_per_w = ni // nw
    n_chunks = 4
    chunk = b_per_w // n_chunks
    mesh = plsc.VectorSubcoreMesh(core_axis_name="c", subcore_axis_name="s")

    @functools.partial(
        pl.kernel,
        out_type=jax.ShapeDtypeStruct((ni, E), jnp.float32),
        mesh=mesh,
        scratch_types=[
            [pltpu.VMEM((chunk,), jnp.int32) for _ in range(n_chunks)],
            pltpu.VMEM((chunk, E), jnp.float32),
            pltpu.SemaphoreType.DMA,
        ],
    )
    def k(table_hbm, idx_hbm, out_hbm, idx_vs, rows_v, sem):
        wid = lax.axis_index("s") * nc + lax.axis_index("c")
        base = wid * b_per_w
        for c in range(n_chunks):
            pltpu.sync_copy(idx_hbm.at[pl.ds(base + c * chunk, chunk)],
                            idx_vs[c])
            pltpu.async_copy(table_hbm.at[idx_vs[c]], rows_v, sem).wait()
            pltpu.sync_copy(rows_v,
                            out_hbm.at[pl.ds(base + c * chunk, chunk)])

    return k(table, idx)


def _dot(a, b):
    return jnp.dot(a.astype(jnp.bfloat16), b,
                   preferred_element_type=jnp.float32)


def _mm_body(a_ref, b_ref, bias_ref, o_ref):
    o_ref[...] = _dot(a_ref[...], b_ref[...]) + bias_ref[...]


def _mm_bias(a, w, bias, bm):
    m, k = a.shape
    n = w.shape[1]
    return pl.pallas_call(
        _mm_body,
        grid=(m // bm,),
        in_specs=[
            pl.BlockSpec((bm, k), lambda i: (i, 0)),
            pl.BlockSpec((k, n), lambda i: (0, 0)),
            pl.BlockSpec((1, n), lambda i: (0, 0)),
        ],
        out_specs=pl.BlockSpec((bm, n), lambda i: (i, 0)),
        out_shape=jax.ShapeDtypeStruct((m, n), jnp.float32),
    )(a, w, bias)


def _mm2_body(a1_ref, b1_ref, a2_ref, b2_ref, bias_ref, o_ref):
    o_ref[...] = (_dot(a1_ref[...], b1_ref[...])
                  + _dot(a2_ref[...], b2_ref[...])
                  + bias_ref[...])


def _mm2_bias(a1, w1, a2, w2, bias, bm):
    m, k = a1.shape
    n = w1.shape[1]
    return pl.pallas_call(
        _mm2_body,
        grid=(m // bm,),
        in_specs=[
            pl.BlockSpec((bm, k), lambda i: (i, 0)),
            pl.BlockSpec((k, n), lambda i: (0, 0)),
            pl.BlockSpec((bm, k), lambda i: (i, 0)),
            pl.BlockSpec((k, n), lambda i: (0, 0)),
            pl.BlockSpec((1, n), lambda i: (0, 0)),
        ],
        out_specs=pl.BlockSpec((bm, n), lambda i: (i, 0)),
        out_shape=jax.ShapeDtypeStruct((m, n), jnp.float32),
    )(a1, w1, a2, w2, bias)


def _lstm_cell(g, h_s, c_s):
    i = jax.nn.sigmoid(g[:, :H])
    f = jax.nn.sigmoid(g[:, H:2 * H])
    gg = jnp.tanh(g[:, 2 * H:3 * H])
    o = jax.nn.sigmoid(g[:, 3 * H:])
    c = f * c_s[...] + i * gg
    h = o * jnp.tanh(c)
    c_s[...] = c
    h_s[...] = h
    return h


U_L = 4


def _lstm0_body(igf_ref, igb_ref, wf_ref, wb_ref, hsf_ref, hsb_ref,
                hf, cf, hb, cb):
    t = pl.program_id(0)

    @pl.when(t == 0)
    def _():
        hf[...] = jnp.zeros_like(hf)
        cf[...] = jnp.zeros_like(cf)
        hb[...] = jnp.zeros_like(hb)
        cb[...] = jnp.zeros_like(cb)

    for j in range(U_L):
        gf = igf_ref[j] + _dot(hf[...], wf_ref[...])
        gb = igb_ref[U_L - 1 - j] + _dot(hb[...], wb_ref[...])
        hsf_ref[j] = _lstm_cell(gf, hf, cf)
        hsb_ref[U_L - 1 - j] = _lstm_cell(gb, hb, cb)


def _lstm_layer0(ig, wf_t, wb_t):
    ng = L // U_L
    return pl.pallas_call(
        _lstm0_body,
        grid=(ng,),
        in_specs=[
            pl.BlockSpec((U_L, N_SEQ, 4 * H), lambda t: (t, 0, 0)),
            pl.BlockSpec((U_L, N_SEQ, 4 * H), lambda t: (ng - 1 - t, 0, 1)),
            pl.BlockSpec((H, 4 * H), lambda t: (0, 0)),
            pl.BlockSpec((H, 4 * H), lambda t: (0, 0)),
        ],
        out_specs=[
            pl.BlockSpec((U_L, N_SEQ, H), lambda t: (t, 0, 0)),
            pl.BlockSpec((U_L, N_SEQ, H), lambda t: (ng - 1 - t, 0, 0)),
        ],
        out_shape=[
            jax.ShapeDtypeStruct((L, N_SEQ, H), jnp.float32),
            jax.ShapeDtypeStruct((L, N_SEQ, H), jnp.float32),
        ],
        scratch_shapes=[pltpu.VMEM((N_SEQ, H), jnp.float32)] * 4,
    )(ig, ig, wf_t, wb_t)


def _lstm1_body(igf_ref, igb_ref, wf_ref, wb_ref, htf_ref, htb_ref,
                hf, cf, hb, cb):
    t = pl.program_id(0)

    @pl.when(t == 0)
    def _():
        hf[...] = jnp.zeros_like(hf)
        cf[...] = jnp.zeros_like(cf)
        hb[...] = jnp.zeros_like(hb)
        cb[...] = jnp.zeros_like(cb)

    for j in range(U_L):
        gf = igf_ref[j] + _dot(hf[...], wf_ref[...])
        gb = igb_ref[U_L - 1 - j] + _dot(hb[...], wb_ref[...])
        htf_ref[...] = _lstm_cell(gf, hf, cf)
        htb_ref[...] = _lstm_cell(gb, hb, cb)


def _lstm_layer1(ig, wf_t, wb_t):
    ng = L // U_L
    return pl.pallas_call(
        _lstm1_body,
        grid=(ng,),
        in_specs=[
            pl.BlockSpec((U_L, N_SEQ, 4 * H), lambda t: (t, 0, 0)),
            pl.BlockSpec((U_L, N_SEQ, 4 * H), lambda t: (ng - 1 - t, 0, 1)),
            pl.BlockSpec((H, 4 * H), lambda t: (0, 0)),
            pl.BlockSpec((H, 4 * H), lambda t: (0, 0)),
        ],
        out_specs=[
            pl.BlockSpec((N_SEQ, H), lambda t: (0, 0)),
            pl.BlockSpec((N_SEQ, H), lambda t: (0, 0)),
        ],
        out_shape=[
            jax.ShapeDtypeStruct((N_SEQ, H), jnp.float32),
            jax.ShapeDtypeStruct((N_SEQ, H), jnp.float32),
        ],
        scratch_shapes=[pltpu.VMEM((N_SEQ, H), jnp.float32)] * 4,
    )(ig, ig, wf_t, wb_t)


def _bil_body(bin_ref, btmp_ref, hu8_ref, w_ref, bb_ref,
              sal_ref, os_ref):
    nt = B * T
    t1 = jnp.dot(bin_ref[...], w_ref[...],
                 preferred_element_type=jnp.float32)
    s = jnp.sum(t1 * btmp_ref[...], axis=1, keepdims=True) + bb_ref[0, 0]
    sal = jax.nn.sigmoid(s)
    sal_ref[...] = jnp.broadcast_to(sal, (nt, 128))
    es = jnp.exp(sal)
    row = lax.broadcasted_iota(jnp.int32, (nt, nt), 0)
    col = lax.broadcasted_iota(jnp.int32, (nt, nt), 1)
    g = jnp.where(row // T == col // T, 1.0, 0.0).astype(jnp.float32)
    denom = jnp.dot(g, es, preferred_element_type=jnp.float32)
    w = es / denom
    rowp = lax.broadcasted_iota(jnp.int32, (B, nt), 0)
    colp = lax.broadcasted_iota(jnp.int32, (B, nt), 1)
    p = jnp.where(colp // T == rowp, 1.0, 0.0).astype(jnp.float32)
    ts = jnp.dot(p, w * btmp_ref[...], preferred_element_type=jnp.float32)
    os_ref[...] = jnp.concatenate([hu8_ref[...], ts], axis=1)


def _bilinear(bil_in, bil_tmp, hu8, bil_w0, bil_b):
    nt = B * T
    return pl.pallas_call(
        _bil_body,
        in_specs=[
            pl.BlockSpec((nt, 2 * H), lambda: (0, 0)),
            pl.BlockSpec((nt, 2 * H), lambda: (0, 0)),
            pl.BlockSpec((B, 2 * H), lambda: (0, 0)),
            pl.BlockSpec((2 * H, 2 * H), lambda: (0, 0)),
            pl.BlockSpec(memory_space=pltpu.SMEM),
        ],
        out_specs=[
            pl.BlockSpec((nt, 128), lambda: (0, 0)),
            pl.BlockSpec((B, 4 * H), lambda: (0, 0)),
        ],
        out_shape=[
            jax.ShapeDtypeStruct((nt, 128), jnp.float32),
            jax.ShapeDtypeStruct((B, 4 * H), jnp.float32),
        ],
    )(bil_in, bil_tmp, hu8, bil_w0, bil_b.reshape(1, 1))


U_G = 4


def _gru_body(gi_ref, u_ref, bh_ref, h0_ref, hs_ref, h):
    t = pl.program_id(0)

    @pl.when(t == 0)
    def _():
        h[...] = h0_ref[...]

    hh = 4 * H
    for j in range(U_G):
        hv = h[...]
        gh = _dot(hv, u_ref[...]) + bh_ref[...]
        gi = gi_ref[j]
        r = jax.nn.sigmoid(gi[:, :hh] + gh[:, :hh])
        z = jax.nn.sigmoid(gi[:, hh:2 * hh] + gh[:, hh:2 * hh])
        nn_ = jnp.tanh(gi[:, 2 * hh:] + r * gh[:, 2 * hh:])
        hn = (1.0 - z) * nn_ + z * hv
        h[...] = hn
        hs_ref[j] = hn


def _gru_seq(gi, u_t, bh, h0):
    return pl.pallas_call(
        _gru_body,
        grid=(Lt // U_G,),
        in_specs=[
            pl.BlockSpec((U_G, B, 12 * H), lambda t: (t, 0, 0)),
            pl.BlockSpec((4 * H, 12 * H), lambda t: (0, 0)),
            pl.BlockSpec((1, 12 * H), lambda t: (0, 0)),
            pl.BlockSpec((B, 4 * H), lambda t: (0, 0)),
        ],
        out_specs=pl.BlockSpec((U_G, B, 4 * H), lambda t: (t, 0, 0)),
        out_shape=jax.ShapeDtypeStruct((Lt, B, 4 * H), jnp.float32),
        scratch_shapes=[pltpu.VMEM((B, 4 * H), jnp.float32)],
    )(gi, u_t, bh, h0)


def _proj_body(a_ref, w_ref, b_ref, o_ref):
    logits = _dot(a_ref[...], w_ref[...]) + b_ref[...]
    m = jnp.max(logits, axis=-1, keepdims=True)
    lse = jnp.log(jnp.sum(jnp.exp(logits - m), axis=-1, keepdims=True)) + m
    o_ref[...] = logits - lse


def _out_proj(flat, w_t, bias):
    bm = 128
    return pl.pallas_call(
        _proj_body,
        grid=(B * Lt // bm,),
        in_specs=[
            pl.BlockSpec((bm, 4 * H), lambda i: (i, 0)),
            pl.BlockSpec((4 * H, V), lambda i: (0, 0)),
            pl.BlockSpec((1, V), lambda i: (0, 0)),
        ],
        out_specs=pl.BlockSpec((bm, V), lambda i: (i, 0)),
        out_shape=jax.ShapeDtypeStruct((B * Lt, V), jnp.float32),
    )(flat, w_t, bias)


def kernel(input_ids, target_ids, template_ids, emb,
           l0f_Wih, l0f_Whh, l0f_bih, l0f_bhh,
           l0b_Wih, l0b_Whh, l0b_bih, l0b_bhh,
           l1f_Wih, l1f_Whh, l1f_bih, l1f_bhh,
           l1b_Wih, l1b_Whh, l1b_bih, l1b_bhh,
           bil_W, bil_b, gru_Wih, gru_Whh, gru_bih, gru_bhh,
           out_W, out_b):
    stacked = jnp.concatenate(
        [input_ids, template_ids.reshape(B * T, L)], 0)
    idx_enc = stacked.T.reshape(-1)
    idx_dec = target_ids.T.reshape(-1)
    idx_all = jnp.concatenate([idx_enc, idx_dec]).astype(jnp.int32)
    rows = _sc_gather(emb, idx_all)
    x_enc = rows[:L * N_SEQ]
    x_dec = rows[L * N_SEQ:]

    b0 = jnp.concatenate([l0f_bih + l0f_bhh, l0b_bih + l0b_bhh]).reshape(1, -1)
    w0 = jnp.concatenate([l0f_Wih.T, l0b_Wih.T],
                         axis=1).astype(jnp.bfloat16)
    ig0 = _mm_bias(x_enc, w0, b0, 1024).reshape(L, N_SEQ, 2 * 4 * H)
    hsf, hsb = _lstm_layer0(ig0, l0f_Whh.T.astype(jnp.bfloat16),
                            l0b_Whh.T.astype(jnp.bfloat16))

    b1 = jnp.concatenate([l1f_bih + l1f_bhh, l1b_bih + l1b_bhh]).reshape(1, -1)
    w1a = jnp.concatenate([l1f_Wih.T[:H], l1b_Wih.T[:H]],
                          axis=1).astype(jnp.bfloat16)
    w1b_ = jnp.concatenate([l1f_Wih.T[H:], l1b_Wih.T[H:]],
                           axis=1).astype(jnp.bfloat16)
    ig1 = _mm2_bias(hsf.reshape(-1, H), w1a, hsb.reshape(-1, H), w1b_,
                    b1, 1024).reshape(L, N_SEQ, 2 * 4 * H)
    htf, htb = _lstm_layer1(ig1, l1f_Whh.T.astype(jnp.bfloat16),
                            l1b_Whh.T.astype(jnp.bfloat16))

    hidden_u = jnp.concatenate([
        jnp.concatenate([htb[1::2], htb[0::2]], axis=1),
        jnp.concatenate([htf[1::2], htf[0::2]], axis=1)], axis=0)
    bil_in = hidden_u[jnp.repeat(jnp.arange(B), T)]
    bil_tmp = hidden_u[B:]
    sal_pad, out_states = _bilinear(bil_in, bil_tmp, hidden_u[:B],
                                    bil_W[0], bil_b)
    sal_b = sal_pad[:, :1].reshape(B, T, 1)

    bgi = gru_bih.reshape(1, -1)
    gi = _mm_bias(x_dec, gru_Wih.T.astype(jnp.bfloat16),
                  bgi, 512).reshape(Lt, B, 12 * H)
    hs = _gru_seq(gi, gru_Whh.T.astype(jnp.bfloat16), gru_bhh.reshape(1, -1),
                  out_states[::-1])

    flat = hs.transpose(1, 0, 2).reshape(B * Lt, 4 * H)
    lp = _out_proj(flat, out_W.T.astype(jnp.bfloat16),
                   out_b.reshape(1, -1))
    response = lp.reshape(B, Lt, V)
    return (sal_b, response)


# speedup vs baseline: 1.8886x; 1.0798x over previous
import functools

import jax
import jax.numpy as jnp
from jax import lax
from jax.experimental import pallas as pl
from jax.experimental.pallas import tpu as pltpu
from jax.experimental.pallas import tpu_sc as plsc

B, T, L, Lt, V, E, H = 8, 4, 512, 256, 8000, 256, 256
N_SEQ = B + B * T


def _sc_gather(table, idx):
    ni = idx.shape[0]
    info = plsc.get_sparse_core_info()
    nc, ns = info.num_cores, info.num_subcores
    nw = nc * ns
    b_per_w = ni // nw
    n_chunks = 4
    chunk = b_per_w // n_chunks
    mesh = plsc.VectorSubcoreMesh(core_axis_name="c", subcore_axis_name="s")

    @functools.partial(
        pl.kernel,
        out_type=jax.ShapeDtypeStruct((ni, E), jnp.float32),
        mesh=mesh,
        scratch_types=[
            [pltpu.VMEM((chunk,), jnp.int32) for _ in range(n_chunks)],
            pltpu.VMEM((chunk, E), jnp.float32),
            pltpu.SemaphoreType.DMA,
        ],
    )
    def k(table_hbm, idx_hbm, out_hbm, idx_vs, rows_v, sem):
        wid = lax.axis_index("s") * nc + lax.axis_index("c")
        base = wid * b_per_w
        for c in range(n_chunks):
            pltpu.sync_copy(idx_hbm.at[pl.ds(base + c * chunk, chunk)],
                            idx_vs[c])
            pltpu.async_copy(table_hbm.at[idx_vs[c]], rows_v, sem).wait()
            pltpu.sync_copy(rows_v,
                            out_hbm.at[pl.ds(base + c * chunk, chunk)])

    return k(table, idx)


def _dot(a, b):
    return jnp.dot(a.astype(jnp.bfloat16), b,
                   preferred_element_type=jnp.float32)


def _mm_body(a_ref, b_ref, bias_ref, o_ref):
    o_ref[...] = _dot(a_ref[...], b_ref[...]) + bias_ref[...]


def _mm_bias(a, w, bias, bm):
    m, k = a.shape
    n = w.shape[1]
    return pl.pallas_call(
        _mm_body,
        grid=(m // bm,),
        in_specs=[
            pl.BlockSpec((bm, k), lambda i: (i, 0)),
            pl.BlockSpec((k, n), lambda i: (0, 0)),
            pl.BlockSpec((1, n), lambda i: (0, 0)),
        ],
        out_specs=pl.BlockSpec((bm, n), lambda i: (i, 0)),
        out_shape=jax.ShapeDtypeStruct((m, n), jnp.float32),
    )(a, w, bias)


def _mm2_body(a1_ref, b1_ref, a2_ref, b2_ref, bias_ref, o_ref):
    o_ref[...] = (_dot(a1_ref[...], b1_ref[...])
                  + _dot(a2_ref[...], b2_ref[...])
                  + bias_ref[...])


def _mm2_bias(a1, w1, a2, w2, bias, bm):
    m, k = a1.shape
    n = w1.shape[1]
    return pl.pallas_call(
        _mm2_body,
        grid=(m // bm,),
        in_specs=[
            pl.BlockSpec((bm, k), lambda i: (i, 0)),
            pl.BlockSpec((k, n), lambda i: (0, 0)),
            pl.BlockSpec((bm, k), lambda i: (i, 0)),
            pl.BlockSpec((k, n), lambda i: (0, 0)),
            pl.BlockSpec((1, n), lambda i: (0, 0)),
        ],
        out_specs=pl.BlockSpec((bm, n), lambda i: (i, 0)),
        out_shape=jax.ShapeDtypeStruct((m, n), jnp.float32),
    )(a1, w1, a2, w2, bias)


def _lstm_cell(g, h_s, c_s):
    i = jax.nn.sigmoid(g[:, :H])
    f = jax.nn.sigmoid(g[:, H:2 * H])
    gg = jnp.tanh(g[:, 2 * H:3 * H])
    o = jax.nn.sigmoid(g[:, 3 * H:])
    c = f * c_s[...] + i * gg
    h = o * jnp.tanh(c)
    c_s[...] = c
    h_s[...] = h
    return h


U_L = 8


def _lstm0_body(igf_ref, igb_ref, wf_ref, wb_ref, hsf_ref, hsb_ref,
                hf, cf, hb, cb):
    t = pl.program_id(0)

    @pl.when(t == 0)
    def _():
        hf[...] = jnp.zeros_like(hf)
        cf[...] = jnp.zeros_like(cf)
        hb[...] = jnp.zeros_like(hb)
        cb[...] = jnp.zeros_like(cb)

    for j in range(U_L):
        gf = igf_ref[j] + _dot(hf[...], wf_ref[...])
        gb = igb_ref[U_L - 1 - j] + _dot(hb[...], wb_ref[...])
        hsf_ref[j] = _lstm_cell(gf, hf, cf)
        hsb_ref[U_L - 1 - j] = _lstm_cell(gb, hb, cb)


def _lstm_layer0(ig, wf_t, wb_t):
    ng = L // U_L
    return pl.pallas_call(
        _lstm0_body,
        grid=(ng,),
        in_specs=[
            pl.BlockSpec((U_L, N_SEQ, 4 * H), lambda t: (t, 0, 0)),
            pl.BlockSpec((U_L, N_SEQ, 4 * H), lambda t: (ng - 1 - t, 0, 1)),
            pl.BlockSpec((H, 4 * H), lambda t: (0, 0)),
            pl.BlockSpec((H, 4 * H), lambda t: (0, 0)),
        ],
        out_specs=[
            pl.BlockSpec((U_L, N_SEQ, H), lambda t: (t, 0, 0)),
            pl.BlockSpec((U_L, N_SEQ, H), lambda t: (ng - 1 - t, 0, 0)),
        ],
        out_shape=[
            jax.ShapeDtypeStruct((L, N_SEQ, H), jnp.float32),
            jax.ShapeDtypeStruct((L, N_SEQ, H), jnp.float32),
        ],
        scratch_shapes=[pltpu.VMEM((N_SEQ, H), jnp.float32)] * 4,
    )(ig, ig, wf_t, wb_t)


def _lstm1_body(igf_ref, igb_ref, wf_ref, wb_ref, htf_ref, htb_ref,
                hf, cf, hb, cb):
    t = pl.program_id(0)

    @pl.when(t == 0)
    def _():
        hf[...] = jnp.zeros_like(hf)
        cf[...] = jnp.zeros_like(cf)
        hb[...] = jnp.zeros_like(hb)
        cb[...] = jnp.zeros_like(cb)

    for j in range(U_L):
        gf = igf_ref[j] + _dot(hf[...], wf_ref[...])
        gb = igb_ref[U_L - 1 - j] + _dot(hb[...], wb_ref[...])
        htf_ref[...] = _lstm_cell(gf, hf, cf)
        htb_ref[...] = _lstm_cell(gb, hb, cb)


def _lstm_layer1(ig, wf_t, wb_t):
    ng = L // U_L
    return pl.pallas_call(
        _lstm1_body,
        grid=(ng,),
        in_specs=[
            pl.BlockSpec((U_L, N_SEQ, 4 * H), lambda t: (t, 0, 0)),
            pl.BlockSpec((U_L, N_SEQ, 4 * H), lambda t: (ng - 1 - t, 0, 1)),
            pl.BlockSpec((H, 4 * H), lambda t: (0, 0)),
            pl.BlockSpec((H, 4 * H), lambda t: (0, 0)),
        ],
        out_specs=[
            pl.BlockSpec((N_SEQ, H), lambda t: (0, 0)),
            pl.BlockSpec((N_SEQ, H), lambda t: (0, 0)),
        ],
        out_shape=[
            jax.ShapeDtypeStruct((N_SEQ, H), jnp.float32),
            jax.ShapeDtypeStruct((N_SEQ, H), jnp.float32),
        ],
        scratch_shapes=[pltpu.VMEM((N_SEQ, H), jnp.float32)] * 4,
    )(ig, ig, wf_t, wb_t)


def _bil_body(bin_ref, btmp_ref, hu8_ref, w_ref, bb_ref,
              sal_ref, os_ref):
    nt = B * T
    t1 = jnp.dot(bin_ref[...], w_ref[...],
                 preferred_element_type=jnp.float32)
    s = jnp.sum(t1 * btmp_ref[...], axis=1, keepdims=True) + bb_ref[0, 0]
    sal = jax.nn.sigmoid(s)
    sal_ref[...] = jnp.broadcast_to(sal, (nt, 128))
    es = jnp.exp(sal)
    row = lax.broadcasted_iota(jnp.int32, (nt, nt), 0)
    col = lax.broadcasted_iota(jnp.int32, (nt, nt), 1)
    g = jnp.where(row // T == col // T, 1.0, 0.0).astype(jnp.float32)
    denom = jnp.dot(g, es, preferred_element_type=jnp.float32)
    w = es / denom
    rowp = lax.broadcasted_iota(jnp.int32, (B, nt), 0)
    colp = lax.broadcasted_iota(jnp.int32, (B, nt), 1)
    p = jnp.where(colp // T == rowp, 1.0, 0.0).astype(jnp.float32)
    ts = jnp.dot(p, w * btmp_ref[...], preferred_element_type=jnp.float32)
    os_ref[...] = jnp.concatenate([hu8_ref[...], ts], axis=1)


def _bilinear(bil_in, bil_tmp, hu8, bil_w0, bil_b):
    nt = B * T
    return pl.pallas_call(
        _bil_body,
        in_specs=[
            pl.BlockSpec((nt, 2 * H), lambda: (0, 0)),
            pl.BlockSpec((nt, 2 * H), lambda: (0, 0)),
            pl.BlockSpec((B, 2 * H), lambda: (0, 0)),
            pl.BlockSpec((2 * H, 2 * H), lambda: (0, 0)),
            pl.BlockSpec(memory_space=pltpu.SMEM),
        ],
        out_specs=[
            pl.BlockSpec((nt, 128), lambda: (0, 0)),
            pl.BlockSpec((B, 4 * H), lambda: (0, 0)),
        ],
        out_shape=[
            jax.ShapeDtypeStruct((nt, 128), jnp.float32),
            jax.ShapeDtypeStruct((B, 4 * H), jnp.float32),
        ],
    )(bil_in, bil_tmp, hu8, bil_w0, bil_b.reshape(1, 1))


U_G = 8


def _gru_body(gi_ref, u_ref, bh_ref, h0_ref, hs_ref, h):
    t = pl.program_id(0)

    @pl.when(t == 0)
    def _():
        h[...] = h0_ref[...]

    hh = 4 * H
    for j in range(U_G):
        hv = h[...]
        gh = _dot(hv, u_ref[...]) + bh_ref[...]
        gi = gi_ref[j]
        r = jax.nn.sigmoid(gi[:, :hh] + gh[:, :hh])
        z = jax.nn.sigmoid(gi[:, hh:2 * hh] + gh[:, hh:2 * hh])
        nn_ = jnp.tanh(gi[:, 2 * hh:] + r * gh[:, 2 * hh:])
        hn = (1.0 - z) * nn_ + z * hv
        h[...] = hn
        hs_ref[j] = hn


def _gru_seq(gi, u_t, bh, h0):
    return pl.pallas_call(
        _gru_body,
        grid=(Lt // U_G,),
        in_specs=[
            pl.BlockSpec((U_G, B, 12 * H), lambda t: (t, 0, 0)),
            pl.BlockSpec((4 * H, 12 * H), lambda t: (0, 0)),
            pl.BlockSpec((1, 12 * H), lambda t: (0, 0)),
            pl.BlockSpec((B, 4 * H), lambda t: (0, 0)),
        ],
        out_specs=pl.BlockSpec((U_G, B, 4 * H), lambda t: (t, 0, 0)),
        out_shape=jax.ShapeDtypeStruct((Lt, B, 4 * H), jnp.float32),
        scratch_shapes=[pltpu.VMEM((B, 4 * H), jnp.float32)],
    )(gi, u_t, bh, h0)


def _proj_body(a_ref, w_ref, b_ref, o_ref):
    logits = _dot(a_ref[...], w_ref[...]) + b_ref[...]
    m = jnp.max(logits, axis=-1, keepdims=True)
    lse = jnp.log(jnp.sum(jnp.exp(logits - m), axis=-1, keepdims=True)) + m
    o_ref[...] = logits - lse


def _out_proj(flat, w_t, bias):
    bm = 128
    return pl.pallas_call(
        _proj_body,
        grid=(B * Lt // bm,),
        in_specs=[
            pl.BlockSpec((bm, 4 * H), lambda i: (i, 0)),
            pl.BlockSpec((4 * H, V), lambda i: (0, 0)),
            pl.BlockSpec((1, V), lambda i: (0, 0)),
        ],
        out_specs=pl.BlockSpec((bm, V), lambda i: (i, 0)),
        out_shape=jax.ShapeDtypeStruct((B * Lt, V), jnp.float32),
    )(flat, w_t, bias)


def kernel(input_ids, target_ids, template_ids, emb,
           l0f_Wih, l0f_Whh, l0f_bih, l0f_bhh,
           l0b_Wih, l0b_Whh, l0b_bih, l0b_bhh,
           l1f_Wih, l1f_Whh, l1f_bih, l1f_bhh,
           l1b_Wih, l1b_Whh, l1b_bih, l1b_bhh,
           bil_W, bil_b, gru_Wih, gru_Whh, gru_bih, gru_bhh,
           out_W, out_b):
    stacked = jnp.concatenate(
        [input_ids, template_ids.reshape(B * T, L)], 0)
    idx_enc = stacked.T.reshape(-1)
    idx_dec = target_ids.T.reshape(-1)
    idx_all = jnp.concatenate([idx_enc, idx_dec]).astype(jnp.int32)
    rows = _sc_gather(emb, idx_all)
    x_enc = rows[:L * N_SEQ]
    x_dec = rows[L * N_SEQ:]

    b0 = jnp.concatenate([l0f_bih + l0f_bhh, l0b_bih + l0b_bhh]).reshape(1, -1)
    w0 = jnp.concatenate([l0f_Wih.T, l0b_Wih.T],
                         axis=1).astype(jnp.bfloat16)
    ig0 = _mm_bias(x_enc, w0, b0, 1024).reshape(L, N_SEQ, 2 * 4 * H)
    hsf, hsb = _lstm_layer0(ig0, l0f_Whh.T.astype(jnp.bfloat16),
                            l0b_Whh.T.astype(jnp.bfloat16))

    b1 = jnp.concatenate([l1f_bih + l1f_bhh, l1b_bih + l1b_bhh]).reshape(1, -1)
    w1a = jnp.concatenate([l1f_Wih.T[:H], l1b_Wih.T[:H]],
                          axis=1).astype(jnp.bfloat16)
    w1b_ = jnp.concatenate([l1f_Wih.T[H:], l1b_Wih.T[H:]],
                           axis=1).astype(jnp.bfloat16)
    ig1 = _mm2_bias(hsf.reshape(-1, H), w1a, hsb.reshape(-1, H), w1b_,
                    b1, 1024).reshape(L, N_SEQ, 2 * 4 * H)
    htf, htb = _lstm_layer1(ig1, l1f_Whh.T.astype(jnp.bfloat16),
                            l1b_Whh.T.astype(jnp.bfloat16))

    hidden_u = jnp.concatenate([
        jnp.concatenate([htb[1::2], htb[0::2]], axis=1),
        jnp.concatenate([htf[1::2], htf[0::2]], axis=1)], axis=0)
    bil_in = hidden_u[jnp.repeat(jnp.arange(B), T)]
    bil_tmp = hidden_u[B:]
    sal_pad, out_states = _bilinear(bil_in, bil_tmp, hidden_u[:B],
                                    bil_W[0], bil_b)
    sal_b = sal_pad[:, :1].reshape(B, T, 1)

    bgi = gru_bih.reshape(1, -1)
    gi = _mm_bias(x_dec, gru_Wih.T.astype(jnp.bfloat16),
                  bgi, 512).reshape(Lt, B, 12 * H)
    hs = _gru_seq(gi, gru_Whh.T.astype(jnp.bfloat16), gru_bhh.reshape(1, -1),
                  out_states[::-1])

    flat = hs.transpose(1, 0, 2).reshape(B * Lt, 4 * H)
    lp = _out_proj(flat, out_W.T.astype(jnp.bfloat16),
                   out_b.reshape(1, -1))
    response = lp.reshape(B, Lt, V)
    return (sal_b, response)


# speedup vs baseline: 2.0146x; 1.0667x over previous
import functools

import jax
import jax.numpy as jnp
from jax import lax
from jax.experimental import pallas as pl
from jax.experimental.pallas import tpu as pltpu
from jax.experimental.pallas import tpu_sc as plsc

B, T, L, Lt, V, E, H = 8, 4, 512, 256, 8000, 256, 256
N_SEQ = B + B * T


def _sc_gather(table, idx):
    ni = idx.shape[0]
    info = plsc.get_sparse_core_info()
    nc, ns = info.num_cores, info.num_subcores
    nw = nc * ns
    b_per_w = ni // nw
    n_chunks = 4
    chunk = b_per_w // n_chunks
    mesh = plsc.VectorSubcoreMesh(core_axis_name="c", subcore_axis_name="s")

    @functools.partial(
        pl.kernel,
        out_type=jax.ShapeDtypeStruct((ni, E), jnp.float32),
        mesh=mesh,
        scratch_types=[
            [pltpu.VMEM((chunk,), jnp.int32) for _ in range(n_chunks)],
            pltpu.VMEM((chunk, E), jnp.float32),
            pltpu.SemaphoreType.DMA,
        ],
    )
    def k(table_hbm, idx_hbm, out_hbm, idx_vs, rows_v, sem):
        wid = lax.axis_index("s") * nc + lax.axis_index("c")
        base = wid * b_per_w
        for c in range(n_chunks):
            pltpu.sync_copy(idx_hbm.at[pl.ds(base + c * chunk, chunk)],
                            idx_vs[c])
            pltpu.async_copy(table_hbm.at[idx_vs[c]], rows_v, sem).wait()
            pltpu.sync_copy(rows_v,
                            out_hbm.at[pl.ds(base + c * chunk, chunk)])

    return k(table, idx)


def _dot(a, b):
    return jnp.dot(a.astype(jnp.bfloat16), b,
                   preferred_element_type=jnp.float32)


def _mm_body(a_ref, b_ref, bias_ref, o_ref):
    o_ref[...] = (_dot(a_ref[...], b_ref[...])
                  + bias_ref[...]).astype(o_ref.dtype)


def _mm_bias(a, w, bias, bm):
    m, k = a.shape
    n = w.shape[1]
    return pl.pallas_call(
        _mm_body,
        grid=(m // bm,),
        in_specs=[
            pl.BlockSpec((bm, k), lambda i: (i, 0)),
            pl.BlockSpec((k, n), lambda i: (0, 0)),
            pl.BlockSpec((1, n), lambda i: (0, 0)),
        ],
        out_specs=pl.BlockSpec((bm, n), lambda i: (i, 0)),
        out_shape=jax.ShapeDtypeStruct((m, n), jnp.bfloat16),
    )(a, w, bias)


def _mm2_body(a1_ref, b1_ref, a2_ref, b2_ref, bias_ref, o_ref):
    o_ref[...] = (_dot(a1_ref[...], b1_ref[...])
                  + _dot(a2_ref[...], b2_ref[...])
                  + bias_ref[...]).astype(o_ref.dtype)


def _mm2_bias(a1, w1, a2, w2, bias, bm):
    m, k = a1.shape
    n = w1.shape[1]
    return pl.pallas_call(
        _mm2_body,
        grid=(m // bm,),
        in_specs=[
            pl.BlockSpec((bm, k), lambda i: (i, 0)),
            pl.BlockSpec((k, n), lambda i: (0, 0)),
            pl.BlockSpec((bm, k), lambda i: (i, 0)),
            pl.BlockSpec((k, n), lambda i: (0, 0)),
            pl.BlockSpec((1, n), lambda i: (0, 0)),
        ],
        out_specs=pl.BlockSpec((bm, n), lambda i: (i, 0)),
        out_shape=jax.ShapeDtypeStruct((m, n), jnp.bfloat16),
    )(a1, w1, a2, w2, bias)


def _lstm_cell(g, h_s, c_s):
    i = jax.nn.sigmoid(g[:, :H])
    f = jax.nn.sigmoid(g[:, H:2 * H])
    gg = jnp.tanh(g[:, 2 * H:3 * H])
    o = jax.nn.sigmoid(g[:, 3 * H:])
    c = f * c_s[...] + i * gg
    h = o * jnp.tanh(c)
    c_s[...] = c
    h_s[...] = h
    return h


U_L = 8


def _lstm0_body(igf_ref, igb_ref, wf_ref, wb_ref, hsf_ref, hsb_ref,
                hf, cf, hb, cb):
    t = pl.program_id(0)

    @pl.when(t == 0)
    def _():
        hf[...] = jnp.zeros_like(hf)
        cf[...] = jnp.zeros_like(cf)
        hb[...] = jnp.zeros_like(hb)
        cb[...] = jnp.zeros_like(cb)

    for j in range(U_L):
        gf = igf_ref[j] + _dot(hf[...], wf_ref[...])
        gb = igb_ref[U_L - 1 - j] + _dot(hb[...], wb_ref[...])
        hsf_ref[j] = _lstm_cell(gf, hf, cf).astype(hsf_ref.dtype)
        hsb_ref[U_L - 1 - j] = _lstm_cell(gb, hb, cb).astype(hsb_ref.dtype)


def _lstm_layer0(ig, wf_t, wb_t):
    ng = L // U_L
    return pl.pallas_call(
        _lstm0_body,
        grid=(ng,),
        in_specs=[
            pl.BlockSpec((U_L, N_SEQ, 4 * H), lambda t: (t, 0, 0)),
            pl.BlockSpec((U_L, N_SEQ, 4 * H), lambda t: (ng - 1 - t, 0, 1)),
            pl.BlockSpec((H, 4 * H), lambda t: (0, 0)),
            pl.BlockSpec((H, 4 * H), lambda t: (0, 0)),
        ],
        out_specs=[
            pl.BlockSpec((U_L, N_SEQ, H), lambda t: (t, 0, 0)),
            pl.BlockSpec((U_L, N_SEQ, H), lambda t: (ng - 1 - t, 0, 0)),
        ],
        out_shape=[
            jax.ShapeDtypeStruct((L, N_SEQ, H), jnp.bfloat16),
            jax.ShapeDtypeStruct((L, N_SEQ, H), jnp.bfloat16),
        ],
        scratch_shapes=[pltpu.VMEM((N_SEQ, H), jnp.float32)] * 4,
    )(ig, ig, wf_t, wb_t)


def _lstm1_body(igf_ref, igb_ref, wf_ref, wb_ref, htf_ref, htb_ref,
                hf, cf, hb, cb):
    t = pl.program_id(0)

    @pl.when(t == 0)
    def _():
        hf[...] = jnp.zeros_like(hf)
        cf[...] = jnp.zeros_like(cf)
        hb[...] = jnp.zeros_like(hb)
        cb[...] = jnp.zeros_like(cb)

    for j in range(U_L):
        gf = igf_ref[j] + _dot(hf[...], wf_ref[...])
        gb = igb_ref[U_L - 1 - j] + _dot(hb[...], wb_ref[...])
        htf_ref[...] = _lstm_cell(gf, hf, cf)
        htb_ref[...] = _lstm_cell(gb, hb, cb)


def _lstm_layer1(ig, wf_t, wb_t):
    ng = L // U_L
    return pl.pallas_call(
        _lstm1_body,
        grid=(ng,),
        in_specs=[
            pl.BlockSpec((U_L, N_SEQ, 4 * H), lambda t: (t, 0, 0)),
            pl.BlockSpec((U_L, N_SEQ, 4 * H), lambda t: (ng - 1 - t, 0, 1)),
            pl.BlockSpec((H, 4 * H), lambda t: (0, 0)),
            pl.BlockSpec((H, 4 * H), lambda t: (0, 0)),
        ],
        out_specs=[
            pl.BlockSpec((N_SEQ, H), lambda t: (0, 0)),
            pl.BlockSpec((N_SEQ, H), lambda t: (0, 0)),
        ],
        out_shape=[
            jax.ShapeDtypeStruct((N_SEQ, H), jnp.float32),
            jax.ShapeDtypeStruct((N_SEQ, H), jnp.float32),
        ],
        scratch_shapes=[pltpu.VMEM((N_SEQ, H), jnp.float32)] * 4,
    )(ig, ig, wf_t, wb_t)


def _bil_body(bin_ref, btmp_ref, hu8_ref, w_ref, bb_ref,
              sal_ref, os_ref):
    nt = B * T
    t1 = jnp.dot(bin_ref[...], w_ref[...],
                 preferred_element_type=jnp.float32)
    s = jnp.sum(t1 * btmp_ref[...], axis=1, keepdims=True) + bb_ref[0, 0]
    sal = jax.nn.sigmoid(s)
    sal_ref[...] = jnp.broadcast_to(sal, (nt, 128))
    es = jnp.exp(sal)
    row = lax.broadcasted_iota(jnp.int32, (nt, nt), 0)
    col = lax.broadcasted_iota(jnp.int32, (nt, nt), 1)
    g = jnp.where(row // T == col // T, 1.0, 0.0).astype(jnp.float32)
    denom = jnp.dot(g, es, preferred_element_type=jnp.float32)
    w = es / denom
    rowp = lax.broadcasted_iota(jnp.int32, (B, nt), 0)
    colp = lax.broadcasted_iota(jnp.int32, (B, nt), 1)
    p = jnp.where(colp // T == rowp, 1.0, 0.0).astype(jnp.float32)
    ts = jnp.dot(p, w * btmp_ref[...], preferred_element_type=jnp.float32)
    os_ref[...] = jnp.concatenate([hu8_ref[...], ts], axis=1)


def _bilinear(bil_in, bil_tmp, hu8, bil_w0, bil_b):
    nt = B * T
    return pl.pallas_call(
        _bil_body,
        in_specs=[
            pl.BlockSpec((nt, 2 * H), lambda: (0, 0)),
            pl.BlockSpec((nt, 2 * H), lambda: (0, 0)),
            pl.BlockSpec((B, 2 * H), lambda: (0, 0)),
            pl.BlockSpec((2 * H, 2 * H), lambda: (0, 0)),
            pl.BlockSpec(memory_space=pltpu.SMEM),
        ],
        out_specs=[
            pl.BlockSpec((nt, 128), lambda: (0, 0)),
            pl.BlockSpec((B, 4 * H), lambda: (0, 0)),
        ],
        out_shape=[
            jax.ShapeDtypeStruct((nt, 128), jnp.float32),
            jax.ShapeDtypeStruct((B, 4 * H), jnp.float32),
        ],
    )(bil_in, bil_tmp, hu8, bil_w0, bil_b.reshape(1, 1))


U_G = 8


def _gru_body(gi_ref, u_ref, bh_ref, h0_ref, hs_ref, h):
    t = pl.program_id(0)

    @pl.when(t == 0)
    def _():
        h[...] = h0_ref[...]

    hh = 4 * H
    for j in range(U_G):
        hv = h[...]
        gh = _dot(hv, u_ref[...]) + bh_ref[...]
        gi = gi_ref[j]
        r = jax.nn.sigmoid(gi[:, :hh] + gh[:, :hh])
        z = jax.nn.sigmoid(gi[:, hh:2 * hh] + gh[:, hh:2 * hh])
        nn_ = jnp.tanh(gi[:, 2 * hh:] + r * gh[:, 2 * hh:])
        hn = (1.0 - z) * nn_ + z * hv
        h[...] = hn
        hs_ref[j] = hn.astype(hs_ref.dtype)


def _gru_seq(gi, u_t, bh, h0):
    return pl.pallas_call(
        _gru_body,
        grid=(Lt // U_G,),
        in_specs=[
            pl.BlockSpec((U_G, B, 12 * H), lambda t: (t, 0, 0)),
            pl.BlockSpec((4 * H, 12 * H), lambda t: (0, 0)),
            pl.BlockSpec((1, 12 * H), lambda t: (0, 0)),
            pl.BlockSpec((B, 4 * H), lambda t: (0, 0)),
        ],
        out_specs=pl.BlockSpec((U_G, B, 4 * H), lambda t: (t, 0, 0)),
        out_shape=jax.ShapeDtypeStruct((Lt, B, 4 * H), jnp.bfloat16),
        scratch_shapes=[pltpu.VMEM((B, 4 * H), jnp.float32)],
    )(gi, u_t, bh, h0)


def _proj_body(a_ref, w_ref, b_ref, o_ref):
    logits = _dot(a_ref[...], w_ref[...]) + b_ref[...]
    m = jnp.max(logits, axis=-1, keepdims=True)
    lse = jnp.log(jnp.sum(jnp.exp(logits - m), axis=-1, keepdims=True)) + m
    o_ref[...] = logits - lse


def _out_proj(flat, w_t, bias):
    bm = 128
    return pl.pallas_call(
        _proj_body,
        grid=(B * Lt // bm,),
        in_specs=[
            pl.BlockSpec((bm, 4 * H), lambda i: (i, 0)),
            pl.BlockSpec((4 * H, V), lambda i: (0, 0)),
            pl.BlockSpec((1, V), lambda i: (0, 0)),
        ],
        out_specs=pl.BlockSpec((bm, V), lambda i: (i, 0)),
        out_shape=jax.ShapeDtypeStruct((B * Lt, V), jnp.float32),
    )(flat, w_t, bias)


def kernel(input_ids, target_ids, template_ids, emb,
           l0f_Wih, l0f_Whh, l0f_bih, l0f_bhh,
           l0b_Wih, l0b_Whh, l0b_bih, l0b_bhh,
           l1f_Wih, l1f_Whh, l1f_bih, l1f_bhh,
           l1b_Wih, l1b_Whh, l1b_bih, l1b_bhh,
           bil_W, bil_b, gru_Wih, gru_Whh, gru_bih, gru_bhh,
           out_W, out_b):
    stacked = jnp.concatenate(
        [input_ids, template_ids.reshape(B * T, L)], 0)
    idx_enc = stacked.T.reshape(-1)
    idx_dec = target_ids.T.reshape(-1)
    idx_all = jnp.concatenate([idx_enc, idx_dec]).astype(jnp.int32)
    rows = _sc_gather(emb, idx_all)
    x_enc = rows[:L * N_SEQ]
    x_dec = rows[L * N_SEQ:]

    b0 = jnp.concatenate([l0f_bih + l0f_bhh, l0b_bih + l0b_bhh]).reshape(1, -1)
    w0 = jnp.concatenate([l0f_Wih.T, l0b_Wih.T],
                         axis=1).astype(jnp.bfloat16)
    ig0 = _mm_bias(x_enc, w0, b0, 1024).reshape(L, N_SEQ, 2 * 4 * H)
    hsf, hsb = _lstm_layer0(ig0, l0f_Whh.T.astype(jnp.bfloat16),
                            l0b_Whh.T.astype(jnp.bfloat16))

    b1 = jnp.concatenate([l1f_bih + l1f_bhh, l1b_bih + l1b_bhh]).reshape(1, -1)
    w1a = jnp.concatenate([l1f_Wih.T[:H], l1b_Wih.T[:H]],
                          axis=1).astype(jnp.bfloat16)
    w1b_ = jnp.concatenate([l1f_Wih.T[H:], l1b_Wih.T[H:]],
                           axis=1).astype(jnp.bfloat16)
    ig1 = _mm2_bias(hsf.reshape(-1, H), w1a, hsb.reshape(-1, H), w1b_,
                    b1, 1024).reshape(L, N_SEQ, 2 * 4 * H)
    htf, htb = _lstm_layer1(ig1, l1f_Whh.T.astype(jnp.bfloat16),
                            l1b_Whh.T.astype(jnp.bfloat16))

    hidden_u = jnp.concatenate([
        jnp.concatenate([htb[1::2], htb[0::2]], axis=1),
        jnp.concatenate([htf[1::2], htf[0::2]], axis=1)], axis=0)
    bil_in = hidden_u[jnp.repeat(jnp.arange(B), T)]
    bil_tmp = hidden_u[B:]
    sal_pad, out_states = _bilinear(bil_in, bil_tmp, hidden_u[:B],
                                    bil_W[0], bil_b)
    sal_b = sal_pad[:, :1].reshape(B, T, 1)

    bgi = gru_bih.reshape(1, -1)
    gi = _mm_bias(x_dec, gru_Wih.T.astype(jnp.bfloat16),
                  bgi, 512).reshape(Lt, B, 12 * H)
    hs = _gru_seq(gi, gru_Whh.T.astype(jnp.bfloat16), gru_bhh.reshape(1, -1),
                  out_states[::-1])

    flat = hs.transpose(1, 0, 2).reshape(B * Lt, 4 * H)
    lp = _out_proj(flat, out_W.T.astype(jnp.bfloat16),
                   out_b.reshape(1, -1))
    response = lp.reshape(B, Lt, V)
    return (sal_b, response)


# speedup vs baseline: 2.0515x; 1.0183x over previous
import functools

import jax
import jax.numpy as jnp
from jax import lax
from jax.experimental import pallas as pl
from jax.experimental.pallas import tpu as pltpu
from jax.experimental.pallas import tpu_sc as plsc

B, T, L, Lt, V, E, H = 8, 4, 512, 256, 8000, 256, 256
N_SEQ = B + B * T


def _sc_gather(table, idx):
    ni = idx.shape[0]
    info = plsc.get_sparse_core_info()
    nc, ns = info.num_cores, info.num_subcores
    nw = nc * ns
    b_per_w = ni // nw
    n_chunks = 4
    chunk = b_per_w // n_chunks
    mesh = plsc.VectorSubcoreMesh(core_axis_name="c", subcore_axis_name="s")

    @functools.partial(
        pl.kernel,
        out_type=jax.ShapeDtypeStruct((ni, E), jnp.float32),
        mesh=mesh,
        scratch_types=[
            [pltpu.VMEM((chunk,), jnp.int32) for _ in range(n_chunks)],
            [pltpu.VMEM((chunk, E), jnp.float32) for _ in range(2)],
            [pltpu.SemaphoreType.DMA for _ in range(2)],
        ],
    )
    def k(table_hbm, idx_hbm, out_hbm, idx_vs, rows_vs, sems):
        wid = lax.axis_index("s") * nc + lax.axis_index("c")
        base = wid * b_per_w
        for c in range(n_chunks):
            pltpu.sync_copy(idx_hbm.at[pl.ds(base + c * chunk, chunk)],
                            idx_vs[c])
        cap = pltpu.async_copy(table_hbm.at[idx_vs[0]], rows_vs[0], sems[0])
        for c in range(n_chunks):
            if c + 1 < n_chunks:
                nxt = pltpu.async_copy(table_hbm.at[idx_vs[c + 1]],
                                       rows_vs[(c + 1) % 2],
                                       sems[(c + 1) % 2])
            cap.wait()
            pltpu.sync_copy(rows_vs[c % 2],
                            out_hbm.at[pl.ds(base + c * chunk, chunk)])
            if c + 1 < n_chunks:
                cap = nxt

    return k(table, idx)


def _dot(a, b):
    return jnp.dot(a.astype(jnp.bfloat16), b,
                   preferred_element_type=jnp.float32)


def _mm_body(a_ref, b_ref, bias_ref, o_ref):
    o_ref[...] = (_dot(a_ref[...], b_ref[...])
                  + bias_ref[...]).astype(o_ref.dtype)


def _mm_bias(a, w, bias, bm):
    m, k = a.shape
    n = w.shape[1]
    return pl.pallas_call(
        _mm_body,
        grid=(m // bm,),
        in_specs=[
            pl.BlockSpec((bm, k), lambda i: (i, 0)),
            pl.BlockSpec((k, n), lambda i: (0, 0)),
            pl.BlockSpec((1, n), lambda i: (0, 0)),
        ],
        out_specs=pl.BlockSpec((bm, n), lambda i: (i, 0)),
        out_shape=jax.ShapeDtypeStruct((m, n), jnp.bfloat16),
    )(a, w, bias)


def _mm2_body(a1_ref, b1_ref, a2_ref, b2_ref, bias_ref, o_ref):
    o_ref[...] = (_dot(a1_ref[...], b1_ref[...])
                  + _dot(a2_ref[...], b2_ref[...])
                  + bias_ref[...]).astype(o_ref.dtype)


def _mm2_bias(a1, w1, a2, w2, bias, bm):
    m, k = a1.shape
    n = w1.shape[1]
    return pl.pallas_call(
        _mm2_body,
        grid=(m // bm,),
        in_specs=[
            pl.BlockSpec((bm, k), lambda i: (i, 0)),
            pl.BlockSpec((k, n), lambda i: (0, 0)),
            pl.BlockSpec((bm, k), lambda i: (i, 0)),
            pl.BlockSpec((k, n), lambda i: (0, 0)),
            pl.BlockSpec((1, n), lambda i: (0, 0)),
        ],
        out_specs=pl.BlockSpec((bm, n), lambda i: (i, 0)),
        out_shape=jax.ShapeDtypeStruct((m, n), jnp.bfloat16),
    )(a1, w1, a2, w2, bias)


def _lstm_cell(g, h_s, c_s):
    i = jax.nn.sigmoid(g[:, :H])
    f = jax.nn.sigmoid(g[:, H:2 * H])
    gg = jnp.tanh(g[:, 2 * H:3 * H])
    o = jax.nn.sigmoid(g[:, 3 * H:])
    c = f * c_s[...] + i * gg
    h = o * jnp.tanh(c)
    c_s[...] = c
    h_s[...] = h
    return h


U_L = 32


def _lstm0_body(igf_ref, igb_ref, wf_ref, wb_ref, hsf_ref, hsb_ref,
                hf, cf, hb, cb):
    t = pl.program_id(0)

    @pl.when(t == 0)
    def _():
        hf[...] = jnp.zeros_like(hf)
        cf[...] = jnp.zeros_like(cf)
        hb[...] = jnp.zeros_like(hb)
        cb[...] = jnp.zeros_like(cb)

    for j in range(U_L):
        gf = igf_ref[j] + _dot(hf[...], wf_ref[...])
        gb = igb_ref[U_L - 1 - j] + _dot(hb[...], wb_ref[...])
        hsf_ref[j] = _lstm_cell(gf, hf, cf).astype(hsf_ref.dtype)
        hsb_ref[U_L - 1 - j] = _lstm_cell(gb, hb, cb).astype(hsb_ref.dtype)


def _lstm_layer0(ig, wf_t, wb_t):
    ng = L // U_L
    return pl.pallas_call(
        _lstm0_body,
        grid=(ng,),
        in_specs=[
            pl.BlockSpec((U_L, N_SEQ, 4 * H), lambda t: (t, 0, 0)),
            pl.BlockSpec((U_L, N_SEQ, 4 * H), lambda t: (ng - 1 - t, 0, 1)),
            pl.BlockSpec((H, 4 * H), lambda t: (0, 0)),
            pl.BlockSpec((H, 4 * H), lambda t: (0, 0)),
        ],
        out_specs=[
            pl.BlockSpec((U_L, N_SEQ, H), lambda t: (t, 0, 0)),
            pl.BlockSpec((U_L, N_SEQ, H), lambda t: (ng - 1 - t, 0, 0)),
        ],
        out_shape=[
            jax.ShapeDtypeStruct((L, N_SEQ, H), jnp.bfloat16),
            jax.ShapeDtypeStruct((L, N_SEQ, H), jnp.bfloat16),
        ],
        scratch_shapes=[pltpu.VMEM((N_SEQ, H), jnp.float32)] * 4,
    )(ig, ig, wf_t, wb_t)


def _lstm1_body(igf_ref, igb_ref, wf_ref, wb_ref, htf_ref, htb_ref,
                hf, cf, hb, cb):
    t = pl.program_id(0)

    @pl.when(t == 0)
    def _():
        hf[...] = jnp.zeros_like(hf)
        cf[...] = jnp.zeros_like(cf)
        hb[...] = jnp.zeros_like(hb)
        cb[...] = jnp.zeros_like(cb)

    for j in range(U_L):
        gf = igf_ref[j] + _dot(hf[...], wf_ref[...])
        gb = igb_ref[U_L - 1 - j] + _dot(hb[...], wb_ref[...])
        htf_ref[...] = _lstm_cell(gf, hf, cf)
        htb_ref[...] = _lstm_cell(gb, hb, cb)


def _lstm_layer1(ig, wf_t, wb_t):
    ng = L // U_L
    return pl.pallas_call(
        _lstm1_body,
        grid=(ng,),
        in_specs=[
            pl.BlockSpec((U_L, N_SEQ, 4 * H), lambda t: (t, 0, 0)),
            pl.BlockSpec((U_L, N_SEQ, 4 * H), lambda t: (ng - 1 - t, 0, 1)),
            pl.BlockSpec((H, 4 * H), lambda t: (0, 0)),
            pl.BlockSpec((H, 4 * H), lambda t: (0, 0)),
        ],
        out_specs=[
            pl.BlockSpec((N_SEQ, H), lambda t: (0, 0)),
            pl.BlockSpec((N_SEQ, H), lambda t: (0, 0)),
        ],
        out_shape=[
            jax.ShapeDtypeStruct((N_SEQ, H), jnp.float32),
            jax.ShapeDtypeStruct((N_SEQ, H), jnp.float32),
        ],
        scratch_shapes=[pltpu.VMEM((N_SEQ, H), jnp.float32)] * 4,
    )(ig, ig, wf_t, wb_t)


def _bil_body(bin_ref, btmp_ref, hu8_ref, w_ref, bb_ref,
              sal_ref, os_ref):
    nt = B * T
    t1 = jnp.dot(bin_ref[...], w_ref[...],
                 preferred_element_type=jnp.float32)
    s = jnp.sum(t1 * btmp_ref[...], axis=1, keepdims=True) + bb_ref[0, 0]
    sal = jax.nn.sigmoid(s)
    sal_ref[...] = jnp.broadcast_to(sal, (nt, 128))
    es = jnp.exp(sal)
    row = lax.broadcasted_iota(jnp.int32, (nt, nt), 0)
    col = lax.broadcasted_iota(jnp.int32, (nt, nt), 1)
    g = jnp.where(row // T == col // T, 1.0, 0.0).astype(jnp.float32)
    denom = jnp.dot(g, es, preferred_element_type=jnp.float32)
    w = es / denom
    rowp = lax.broadcasted_iota(jnp.int32, (B, nt), 0)
    colp = lax.broadcasted_iota(jnp.int32, (B, nt), 1)
    p = jnp.where(colp // T == rowp, 1.0, 0.0).astype(jnp.float32)
    ts = jnp.dot(p, w * btmp_ref[...], preferred_element_type=jnp.float32)
    os_ref[...] = jnp.concatenate([hu8_ref[...], ts], axis=1)


def _bilinear(bil_in, bil_tmp, hu8, bil_w0, bil_b):
    nt = B * T
    return pl.pallas_call(
        _bil_body,
        in_specs=[
            pl.BlockSpec((nt, 2 * H), lambda: (0, 0)),
            pl.BlockSpec((nt, 2 * H), lambda: (0, 0)),
            pl.BlockSpec((B, 2 * H), lambda: (0, 0)),
            pl.BlockSpec((2 * H, 2 * H), lambda: (0, 0)),
            pl.BlockSpec(memory_space=pltpu.SMEM),
        ],
        out_specs=[
            pl.BlockSpec((nt, 128), lambda: (0, 0)),
            pl.BlockSpec((B, 4 * H), lambda: (0, 0)),
        ],
        out_shape=[
            jax.ShapeDtypeStruct((nt, 128), jnp.float32),
            jax.ShapeDtypeStruct((B, 4 * H), jnp.float32),
        ],
    )(bil_in, bil_tmp, hu8, bil_w0, bil_b.reshape(1, 1))


U_G = 16


def _gru_body(gi_ref, u_ref, bh_ref, h0_ref, hs_ref, h):
    t = pl.program_id(0)

    @pl.when(t == 0)
    def _():
        h[...] = h0_ref[...]

    hh = 4 * H
    for j in range(U_G):
        hv = h[...]
        gh = _dot(hv, u_ref[...]) + bh_ref[...]
        gi = gi_ref[j]
        r = jax.nn.sigmoid(gi[:, :hh] + gh[:, :hh])
        z = jax.nn.sigmoid(gi[:, hh:2 * hh] + gh[:, hh:2 * hh])
        nn_ = jnp.tanh(gi[:, 2 * hh:] + r * gh[:, 2 * hh:])
        hn = (1.0 - z) * nn_ + z * hv
        h[...] = hn
        hs_ref[j] = hn.astype(hs_ref.dtype)


def _gru_seq(gi, u_t, bh, h0):
    return pl.pallas_call(
        _gru_body,
        grid=(Lt // U_G,),
        in_specs=[
            pl.BlockSpec((U_G, B, 12 * H), lambda t: (t, 0, 0)),
            pl.BlockSpec((4 * H, 12 * H), lambda t: (0, 0)),
            pl.BlockSpec((1, 12 * H), lambda t: (0, 0)),
            pl.BlockSpec((B, 4 * H), lambda t: (0, 0)),
        ],
        out_specs=pl.BlockSpec((U_G, B, 4 * H), lambda t: (t, 0, 0)),
        out_shape=jax.ShapeDtypeStruct((Lt, B, 4 * H), jnp.bfloat16),
        scratch_shapes=[pltpu.VMEM((B, 4 * H), jnp.float32)],
    )(gi, u_t, bh, h0)


def _proj_body(a_ref, w_ref, b_ref, o_ref):
    logits = _dot(a_ref[...], w_ref[...]) + b_ref[...]
    m = jnp.max(logits, axis=-1, keepdims=True)
    lse = jnp.log(jnp.sum(jnp.exp(logits - m), axis=-1, keepdims=True)) + m
    o_ref[...] = logits - lse


def _out_proj(flat, w_t, bias):
    bm = 256
    return pl.pallas_call(
        _proj_body,
        grid=(B * Lt // bm,),
        in_specs=[
            pl.BlockSpec((bm, 4 * H), lambda i: (i, 0)),
            pl.BlockSpec((4 * H, V), lambda i: (0, 0)),
            pl.BlockSpec((1, V), lambda i: (0, 0)),
        ],
        out_specs=pl.BlockSpec((bm, V), lambda i: (i, 0)),
        out_shape=jax.ShapeDtypeStruct((B * Lt, V), jnp.float32),
    )(flat, w_t, bias)


def kernel(input_ids, target_ids, template_ids, emb,
           l0f_Wih, l0f_Whh, l0f_bih, l0f_bhh,
           l0b_Wih, l0b_Whh, l0b_bih, l0b_bhh,
           l1f_Wih, l1f_Whh, l1f_bih, l1f_bhh,
           l1b_Wih, l1b_Whh, l1b_bih, l1b_bhh,
           bil_W, bil_b, gru_Wih, gru_Whh, gru_bih, gru_bhh,
           out_W, out_b):
    stacked = jnp.concatenate(
        [input_ids, template_ids.reshape(B * T, L)], 0)
    idx_enc = stacked.T.reshape(-1)
    idx_dec = target_ids.T.reshape(-1)
    idx_all = jnp.concatenate([idx_enc, idx_dec]).astype(jnp.int32)
    rows = _sc_gather(emb, idx_all)
    x_enc = rows[:L * N_SEQ]
    x_dec = rows[L * N_SEQ:]

    b0 = jnp.concatenate([l0f_bih + l0f_bhh, l0b_bih + l0b_bhh]).reshape(1, -1)
    w0 = jnp.concatenate([l0f_Wih.T, l0b_Wih.T],
                         axis=1).astype(jnp.bfloat16)
    ig0 = _mm_bias(x_enc, w0, b0, 1024).reshape(L, N_SEQ, 2 * 4 * H)
    hsf, hsb = _lstm_layer0(ig0, l0f_Whh.T.astype(jnp.bfloat16),
                            l0b_Whh.T.astype(jnp.bfloat16))

    b1 = jnp.concatenate([l1f_bih + l1f_bhh, l1b_bih + l1b_bhh]).reshape(1, -1)
    w1a = jnp.concatenate([l1f_Wih.T[:H], l1b_Wih.T[:H]],
                          axis=1).astype(jnp.bfloat16)
    w1b_ = jnp.concatenate([l1f_Wih.T[H:], l1b_Wih.T[H:]],
                           axis=1).astype(jnp.bfloat16)
    ig1 = _mm2_bias(hsf.reshape(-1, H), w1a, hsb.reshape(-1, H), w1b_,
                    b1, 1024).reshape(L, N_SEQ, 2 * 4 * H)
    htf, htb = _lstm_layer1(ig1, l1f_Whh.T.astype(jnp.bfloat16),
                            l1b_Whh.T.astype(jnp.bfloat16))

    hidden_u = jnp.concatenate([
        jnp.concatenate([htb[1::2], htb[0::2]], axis=1),
        jnp.concatenate([htf[1::2], htf[0::2]], axis=1)], axis=0)
    bil_in = hidden_u[jnp.repeat(jnp.arange(B), T)]
    bil_tmp = hidden_u[B:]
    sal_pad, out_states = _bilinear(bil_in, bil_tmp, hidden_u[:B],
                                    bil_W[0], bil_b)
    sal_b = sal_pad[:, :1].reshape(B, T, 1)

    bgi = gru_bih.reshape(1, -1)
    gi = _mm_bias(x_dec, gru_Wih.T.astype(jnp.bfloat16),
                  bgi, 512).reshape(Lt, B, 12 * H)
    hs = _gru_seq(gi, gru_Whh.T.astype(jnp.bfloat16), gru_bhh.reshape(1, -1),
                  out_states[::-1])

    flat = hs.transpose(1, 0, 2).reshape(B * Lt, 4 * H)
    lp = _out_proj(flat, out_W.T.astype(jnp.bfloat16),
                   out_b.reshape(1, -1))
    response = lp.reshape(B, Lt, V)
    return (sal_b, response)


# speedup vs baseline: 2.1404x; 1.0433x over previous
import functools

import jax
import jax.numpy as jnp
from jax import lax
from jax.experimental import pallas as pl
from jax.experimental.pallas import tpu as pltpu
from jax.experimental.pallas import tpu_sc as plsc

B, T, L, Lt, V, E, H = 8, 4, 512, 256, 8000, 256, 256
N_SEQ = B + B * T


def _sc_gather(table, idx):
    ni = idx.shape[0]
    info = plsc.get_sparse_core_info()
    nc, ns = info.num_cores, info.num_subcores
    nw = nc * ns
    b_per_w = ni // nw
    n_chunks = 4
    chunk = b_per_w // n_chunks
    mesh = plsc.VectorSubcoreMesh(core_axis_name="c", subcore_axis_name="s")

    @functools.partial(
        pl.kernel,
        out_type=jax.ShapeDtypeStruct((ni, E), jnp.float32),
        mesh=mesh,
        scratch_types=[
            [pltpu.VMEM((chunk,), jnp.int32) for _ in range(n_chunks)],
            [pltpu.VMEM((chunk, E), jnp.float32) for _ in range(2)],
            [pltpu.SemaphoreType.DMA for _ in range(2)],
        ],
    )
    def k(table_hbm, idx_hbm, out_hbm, idx_vs, rows_vs, sems):
        wid = lax.axis_index("s") * nc + lax.axis_index("c")
        base = wid * b_per_w
        for c in range(n_chunks):
            pltpu.sync_copy(idx_hbm.at[pl.ds(base + c * chunk, chunk)],
                            idx_vs[c])
        cap = pltpu.async_copy(table_hbm.at[idx_vs[0]], rows_vs[0], sems[0])
        for c in range(n_chunks):
            if c + 1 < n_chunks:
                nxt = pltpu.async_copy(table_hbm.at[idx_vs[c + 1]],
                                       rows_vs[(c + 1) % 2],
                                       sems[(c + 1) % 2])
            cap.wait()
            pltpu.sync_copy(rows_vs[c % 2],
                            out_hbm.at[pl.ds(base + c * chunk, chunk)])
            if c + 1 < n_chunks:
                cap = nxt

    return k(table, idx)


def _dot(a, b):
    return jnp.dot(a.astype(jnp.bfloat16), b,
                   preferred_element_type=jnp.float32)


def _mm_body(a_ref, b_ref, bias_ref, o_ref):
    o_ref[...] = (_dot(a_ref[...], b_ref[...])
                  + bias_ref[...]).astype(o_ref.dtype)


def _mm_bias(a, w, bias, bm):
    m, k = a.shape
    n = w.shape[1]
    return pl.pallas_call(
        _mm_body,
        grid=(m // bm,),
        in_specs=[
            pl.BlockSpec((bm, k), lambda i: (i, 0)),
            pl.BlockSpec((k, n), lambda i: (0, 0)),
            pl.BlockSpec((1, n), lambda i: (0, 0)),
        ],
        out_specs=pl.BlockSpec((bm, n), lambda i: (i, 0)),
        out_shape=jax.ShapeDtypeStruct((m, n), jnp.bfloat16),
    )(a, w, bias)


def _mm2_body(a1_ref, b1_ref, a2_ref, b2_ref, bias_ref, o_ref):
    o_ref[...] = (_dot(a1_ref[...], b1_ref[...])
                  + _dot(a2_ref[...], b2_ref[...])
                  + bias_ref[...]).astype(o_ref.dtype)


def _mm2_bias(a1, w1, a2, w2, bias, bm):
    m, k = a1.shape
    n = w1.shape[1]
    return pl.pallas_call(
        _mm2_body,
        grid=(m // bm,),
        in_specs=[
            pl.BlockSpec((bm, k), lambda i: (i, 0)),
            pl.BlockSpec((k, n), lambda i: (0, 0)),
            pl.BlockSpec((bm, k), lambda i: (i, 0)),
            pl.BlockSpec((k, n), lambda i: (0, 0)),
            pl.BlockSpec((1, n), lambda i: (0, 0)),
        ],
        out_specs=pl.BlockSpec((bm, n), lambda i: (i, 0)),
        out_shape=jax.ShapeDtypeStruct((m, n), jnp.bfloat16),
    )(a1, w1, a2, w2, bias)


def _lstm_cell(g, h_s, c_s):
    i = jax.nn.sigmoid(g[:, :H])
    f = jax.nn.sigmoid(g[:, H:2 * H])
    gg = jnp.tanh(g[:, 2 * H:3 * H])
    o = jax.nn.sigmoid(g[:, 3 * H:])
    c = f * c_s[...] + i * gg
    h = o * jnp.tanh(c)
    c_s[...] = c
    h_s[...] = h
    return h


U_L = 32


def _lstm0_body(xf_ref, xb_ref, wxf_ref, wxb_ref, bf_ref, bb_ref,
                wf_ref, wb_ref, hsf_ref, hsb_ref,
                hf, cf, hb, cb, gfs, gbs):
    t = pl.program_id(0)

    @pl.when(t == 0)
    def _():
        hf[...] = jnp.zeros_like(hf)
        cf[...] = jnp.zeros_like(cf)
        hb[...] = jnp.zeros_like(hb)
        cb[...] = jnp.zeros_like(cb)

    gfs[...] = _dot(xf_ref[...].reshape(U_L * N_SEQ, E),
                    wxf_ref[...]) + bf_ref[...]
    gbs[...] = _dot(xb_ref[...].reshape(U_L * N_SEQ, E),
                    wxb_ref[...]) + bb_ref[...]
    for j in range(U_L):
        jr = U_L - 1 - j
        gf = gfs[j * N_SEQ:(j + 1) * N_SEQ, :] + _dot(hf[...], wf_ref[...])
        gb = gbs[jr * N_SEQ:(jr + 1) * N_SEQ, :] + _dot(hb[...], wb_ref[...])
        hsf_ref[j] = _lstm_cell(gf, hf, cf).astype(hsf_ref.dtype)
        hsb_ref[jr] = _lstm_cell(gb, hb, cb).astype(hsb_ref.dtype)


def _lstm_layer0(x, wxf, wxb, bf, bb, wf_t, wb_t):
    ng = L // U_L
    return pl.pallas_call(
        _lstm0_body,
        grid=(ng,),
        in_specs=[
            pl.BlockSpec((U_L, N_SEQ, E), lambda t: (t, 0, 0)),
            pl.BlockSpec((U_L, N_SEQ, E), lambda t: (ng - 1 - t, 0, 0)),
            pl.BlockSpec((E, 4 * H), lambda t: (0, 0)),
            pl.BlockSpec((E, 4 * H), lambda t: (0, 0)),
            pl.BlockSpec((1, 4 * H), lambda t: (0, 0)),
            pl.BlockSpec((1, 4 * H), lambda t: (0, 0)),
            pl.BlockSpec((H, 4 * H), lambda t: (0, 0)),
            pl.BlockSpec((H, 4 * H), lambda t: (0, 0)),
        ],
        out_specs=[
            pl.BlockSpec((U_L, N_SEQ, H), lambda t: (t, 0, 0)),
            pl.BlockSpec((U_L, N_SEQ, H), lambda t: (ng - 1 - t, 0, 0)),
        ],
        out_shape=[
            jax.ShapeDtypeStruct((L, N_SEQ, H), jnp.bfloat16),
            jax.ShapeDtypeStruct((L, N_SEQ, H), jnp.bfloat16),
        ],
        scratch_shapes=[pltpu.VMEM((N_SEQ, H), jnp.float32)] * 4
        + [pltpu.VMEM((U_L * N_SEQ, 4 * H), jnp.float32)] * 2,
    )(x, x, wxf, wxb, bf, bb, wf_t, wb_t)


def _lstm1_body(hsf_t_ref, hsb_t_ref, hsf_r_ref, hsb_r_ref,
                w1ft_ref, w1fb_ref, w1bt_ref, w1bb_ref, b1f_ref, b1b_ref,
                wf_ref, wb_ref, htf_ref, htb_ref,
                hf, cf, hb, cb, gfs, gbs):
    t = pl.program_id(0)

    @pl.when(t == 0)
    def _():
        hf[...] = jnp.zeros_like(hf)
        cf[...] = jnp.zeros_like(cf)
        hb[...] = jnp.zeros_like(hb)
        cb[...] = jnp.zeros_like(cb)

    m = U_L * N_SEQ
    gfs[...] = (_dot(hsf_t_ref[...].reshape(m, H), w1ft_ref[...])
                + _dot(hsb_t_ref[...].reshape(m, H), w1fb_ref[...])
                + b1f_ref[...])
    gbs[...] = (_dot(hsf_r_ref[...].reshape(m, H), w1bt_ref[...])
                + _dot(hsb_r_ref[...].reshape(m, H), w1bb_ref[...])
                + b1b_ref[...])
    for j in range(U_L):
        jr = U_L - 1 - j
        gf = gfs[j * N_SEQ:(j + 1) * N_SEQ, :] + _dot(hf[...], wf_ref[...])
        gb = gbs[jr * N_SEQ:(jr + 1) * N_SEQ, :] + _dot(hb[...], wb_ref[...])
        htf_ref[...] = _lstm_cell(gf, hf, cf)
        htb_ref[...] = _lstm_cell(gb, hb, cb)


def _lstm_layer1(hsf, hsb, w1ft, w1fb, w1bt, w1bb, b1f, b1b, wf_t, wb_t):
    ng = L // U_L
    return pl.pallas_call(
        _lstm1_body,
        grid=(ng,),
        in_specs=[
            pl.BlockSpec((U_L, N_SEQ, H), lambda t: (t, 0, 0)),
            pl.BlockSpec((U_L, N_SEQ, H), lambda t: (t, 0, 0)),
            pl.BlockSpec((U_L, N_SEQ, H), lambda t: (ng - 1 - t, 0, 0)),
            pl.BlockSpec((U_L, N_SEQ, H), lambda t: (ng - 1 - t, 0, 0)),
            pl.BlockSpec((H, 4 * H), lambda t: (0, 0)),
            pl.BlockSpec((H, 4 * H), lambda t: (0, 0)),
            pl.BlockSpec((H, 4 * H), lambda t: (0, 0)),
            pl.BlockSpec((H, 4 * H), lambda t: (0, 0)),
            pl.BlockSpec((1, 4 * H), lambda t: (0, 0)),
            pl.BlockSpec((1, 4 * H), lambda t: (0, 0)),
            pl.BlockSpec((H, 4 * H), lambda t: (0, 0)),
            pl.BlockSpec((H, 4 * H), lambda t: (0, 0)),
        ],
        out_specs=[
            pl.BlockSpec((N_SEQ, H), lambda t: (0, 0)),
            pl.BlockSpec((N_SEQ, H), lambda t: (0, 0)),
        ],
        out_shape=[
            jax.ShapeDtypeStruct((N_SEQ, H), jnp.float32),
            jax.ShapeDtypeStruct((N_SEQ, H), jnp.float32),
        ],
        scratch_shapes=[pltpu.VMEM((N_SEQ, H), jnp.float32)] * 4
        + [pltpu.VMEM((U_L * N_SEQ, 4 * H), jnp.float32)] * 2,
    )(hsf, hsb, hsf, hsb, w1ft, w1fb, w1bt, w1bb, b1f, b1b, wf_t, wb_t)


def _bil_body(bin_ref, btmp_ref, hu8_ref, w_ref, bb_ref,
              sal_ref, os_ref):
    nt = B * T
    t1 = jnp.dot(bin_ref[...], w_ref[...],
                 preferred_element_type=jnp.float32)
    s = jnp.sum(t1 * btmp_ref[...], axis=1, keepdims=True) + bb_ref[0, 0]
    sal = jax.nn.sigmoid(s)
    sal_ref[...] = jnp.broadcast_to(sal, (nt, 128))
    es = jnp.exp(sal)
    row = lax.broadcasted_iota(jnp.int32, (nt, nt), 0)
    col = lax.broadcasted_iota(jnp.int32, (nt, nt), 1)
    g = jnp.where(row // T == col // T, 1.0, 0.0).astype(jnp.float32)
    denom = jnp.dot(g, es, preferred_element_type=jnp.float32)
    w = es / denom
    rowp = lax.broadcasted_iota(jnp.int32, (B, nt), 0)
    colp = lax.broadcasted_iota(jnp.int32, (B, nt), 1)
    p = jnp.where(colp // T == rowp, 1.0, 0.0).astype(jnp.float32)
    ts = jnp.dot(p, w * btmp_ref[...], preferred_element_type=jnp.float32)
    os_ref[...] = jnp.concatenate([hu8_ref[...], ts], axis=1)


def _bilinear(bil_in, bil_tmp, hu8, bil_w0, bil_b):
    nt = B * T
    return pl.pallas_call(
        _bil_body,
        in_specs=[
            pl.BlockSpec((nt, 2 * H), lambda: (0, 0)),
            pl.BlockSpec((nt, 2 * H), lambda: (0, 0)),
            pl.BlockSpec((B, 2 * H), lambda: (0, 0)),
            pl.BlockSpec((2 * H, 2 * H), lambda: (0, 0)),
            pl.BlockSpec(memory_space=pltpu.SMEM),
        ],
        out_specs=[
            pl.BlockSpec((nt, 128), lambda: (0, 0)),
            pl.BlockSpec((B, 4 * H), lambda: (0, 0)),
        ],
        out_shape=[
            jax.ShapeDtypeStruct((nt, 128), jnp.float32),
            jax.ShapeDtypeStruct((B, 4 * H), jnp.float32),
        ],
    )(bil_in, bil_tmp, hu8, bil_w0, bil_b.reshape(1, 1))


U_G = 16


def _gru_body(xd_ref, wx_ref, bi_ref, u_ref, bh_ref, h0_ref, hs_ref,
              h, gis):
    t = pl.program_id(0)

    @pl.when(t == 0)
    def _():
        h[...] = h0_ref[...]

    gis[...] = _dot(xd_ref[...].reshape(U_G * B, E), wx_ref[...]) + bi_ref[...]
    hh = 4 * H
    for j in range(U_G):
        hv = h[...]
        gh = _dot(hv, u_ref[...]) + bh_ref[...]
        gi = gis[j * B:(j + 1) * B, :]
        r = jax.nn.sigmoid(gi[:, :hh] + gh[:, :hh])
        z = jax.nn.sigmoid(gi[:, hh:2 * hh] + gh[:, hh:2 * hh])
        nn_ = jnp.tanh(gi[:, 2 * hh:] + r * gh[:, 2 * hh:])
        hn = (1.0 - z) * nn_ + z * hv
        h[...] = hn
        hs_ref[j] = hn.astype(hs_ref.dtype)


def _gru_seq(xd, wx, bi, u_t, bh, h0):
    return pl.pallas_call(
        _gru_body,
        grid=(Lt // U_G,),
        in_specs=[
            pl.BlockSpec((U_G, B, E), lambda t: (t, 0, 0)),
            pl.BlockSpec((E, 12 * H), lambda t: (0, 0)),
            pl.BlockSpec((1, 12 * H), lambda t: (0, 0)),
            pl.BlockSpec((4 * H, 12 * H), lambda t: (0, 0)),
            pl.BlockSpec((1, 12 * H), lambda t: (0, 0)),
            pl.BlockSpec((B, 4 * H), lambda t: (0, 0)),
        ],
        out_specs=pl.BlockSpec((U_G, B, 4 * H), lambda t: (t, 0, 0)),
        out_shape=jax.ShapeDtypeStruct((Lt, B, 4 * H), jnp.bfloat16),
        scratch_shapes=[pltpu.VMEM((B, 4 * H), jnp.float32),
                        pltpu.VMEM((U_G * B, 12 * H), jnp.float32)],
    )(xd, wx, bi, u_t, bh, h0)


def _proj_body(a_ref, w_ref, b_ref, o_ref):
    logits = _dot(a_ref[...], w_ref[...]) + b_ref[...]
    m = jnp.max(logits, axis=-1, keepdims=True)
    lse = jnp.log(jnp.sum(jnp.exp(logits - m), axis=-1, keepdims=True)) + m
    o_ref[...] = logits - lse


def _out_proj(flat, w_t, bias):
    bm = 256
    return pl.pallas_call(
        _proj_body,
        grid=(B * Lt // bm,),
        in_specs=[
            pl.BlockSpec((bm, 4 * H), lambda i: (i, 0)),
            pl.BlockSpec((4 * H, V), lambda i: (0, 0)),
            pl.BlockSpec((1, V), lambda i: (0, 0)),
        ],
        out_specs=pl.BlockSpec((bm, V), lambda i: (i, 0)),
        out_shape=jax.ShapeDtypeStruct((B * Lt, V), jnp.float32),
    )(flat, w_t, bias)


def kernel(input_ids, target_ids, template_ids, emb,
           l0f_Wih, l0f_Whh, l0f_bih, l0f_bhh,
           l0b_Wih, l0b_Whh, l0b_bih, l0b_bhh,
           l1f_Wih, l1f_Whh, l1f_bih, l1f_bhh,
           l1b_Wih, l1b_Whh, l1b_bih, l1b_bhh,
           bil_W, bil_b, gru_Wih, gru_Whh, gru_bih, gru_bhh,
           out_W, out_b):
    stacked = jnp.concatenate(
        [input_ids, template_ids.reshape(B * T, L)], 0)
    idx_enc = stacked.T.reshape(-1)
    idx_dec = target_ids.T.reshape(-1)
    idx_all = jnp.concatenate([idx_enc, idx_dec]).astype(jnp.int32)
    rows = _sc_gather(emb, idx_all)
    x_enc = rows[:L * N_SEQ]
    x_dec = rows[L * N_SEQ:]

    x3 = x_enc.reshape(L, N_SEQ, E)
    bf16 = jnp.bfloat16
    b0f = (l0f_bih + l0f_bhh).reshape(1, -1)
    b0b = (l0b_bih + l0b_bhh).reshape(1, -1)
    hsf, hsb = _lstm_layer0(x3, l0f_Wih.T.astype(bf16), l0b_Wih.T.astype(bf16),
                            b0f, b0b,
                            l0f_Whh.T.astype(bf16), l0b_Whh.T.astype(bf16))
    b1f = (l1f_bih + l1f_bhh).reshape(1, -1)
    b1b = (l1b_bih + l1b_bhh).reshape(1, -1)
    w1f_t = l1f_Wih.T.astype(bf16)
    w1b_t = l1b_Wih.T.astype(bf16)
    htf, htb = _lstm_layer1(hsf, hsb,
                            w1f_t[:H], w1f_t[H:], w1b_t[:H], w1b_t[H:],
                            b1f, b1b,
                            l1f_Whh.T.astype(bf16), l1b_Whh.T.astype(bf16))

    hidden_u = jnp.concatenate([
        jnp.concatenate([htb[1::2], htb[0::2]], axis=1),
        jnp.concatenate([htf[1::2], htf[0::2]], axis=1)], axis=0)
    bil_in = hidden_u[jnp.repeat(jnp.arange(B), T)]
    bil_tmp = hidden_u[B:]
    sal_pad, out_states = _bilinear(bil_in, bil_tmp, hidden_u[:B],
                                    bil_W[0], bil_b)
    sal_b = sal_pad[:, :1].reshape(B, T, 1)

    hs = _gru_seq(x_dec.reshape(Lt, B, E), gru_Wih.T.astype(bf16),
                  gru_bih.reshape(1, -1), gru_Whh.T.astype(bf16),
                  gru_bhh.reshape(1, -1), out_states[::-1])

    flat = hs.transpose(1, 0, 2).reshape(B * Lt, 4 * H)
    lp = _out_proj(flat, out_W.T.astype(jnp.bfloat16),
                   out_b.reshape(1, -1))
    response = lp.reshape(B, Lt, V)
    return (sal_b, response)
